# Initial kernel scaffold; baseline (speedup 1.0000x reference)
import jax
import jax.numpy as jnp
from jax.experimental import pallas as pl


def kernel(x, edge_index, edge_attr, batch, Wl1, bl1, Wr1, br1, We1, att1, bias1, Wl2, bl2, Wr2, br2, We2, att2, bias2, Wlin, blin):
    raise NotImplementedError("write your pallas kernel here")


# speedup vs baseline: 1.0021x; 1.0021x over previous
import functools

import jax
import jax.numpy as jnp
from jax.experimental import pallas as pl
from jax.experimental.pallas import tpu as pltpu

_N = 50000
_E = 800000
_G = 64


def _pool_body(h_ref, batch_ref, wlin_ref, blin_ref, out_ref, acc_ref, cnt_ref):
    i = pl.program_id(0)
    nb = pl.num_programs(0)

    @pl.when(i == 0)
    def _init():
        acc_ref[...] = jnp.zeros_like(acc_ref)
        cnt_ref[...] = jnp.zeros_like(cnt_ref)

    b = batch_ref[0, 0, :]
    onehot = (b[:, None] == jax.lax.iota(jnp.int32, _G)[None, :]).astype(
        jnp.float32)
    acc_ref[...] += jnp.dot(onehot.T, h_ref[...],
                            preferred_element_type=jnp.float32)
    cnt_ref[...] += jnp.sum(onehot, axis=0)[:, None]

    @pl.when(i == nb - 1)
    def _fin():
        pooled = acc_ref[...] / jnp.maximum(cnt_ref[...], 1.0)
        out = jnp.dot(pooled, wlin_ref[...],
                      preferred_element_type=jnp.float32) + blin_ref[...]
        out_ref[...] = jax.nn.sigmoid(out)


def _pool_final(h, batch, Wlin, blin):
    npad = h.shape[0]
    blk = 2000
    grid = npad // blk
    return pl.pallas_call(
        _pool_body,
        grid=(grid,),
        in_specs=[
            pl.BlockSpec((blk, 128), lambda i: (i, 0)),
            pl.BlockSpec((1, 1, blk), lambda i: (i, 0, 0)),
            pl.BlockSpec((128, 1), lambda i: (0, 0)),
            pl.BlockSpec((1, 1), lambda i: (0, 0)),
        ],
        out_specs=pl.BlockSpec((_G, 1), lambda i: (0, 0)),
        out_shape=jax.ShapeDtypeStruct((_G, 1), jnp.float32),
        scratch_shapes=[
            pltpu.VMEM((_G, 128), jnp.float32),
            pltpu.VMEM((_G, 1), jnp.float32),
        ],
    )(h, batch.reshape(npad // blk, 1, blk), Wlin, blin.reshape(1, 1))


def _gatv2_ref(x, src, dst, ea, Wl, bl, Wr, br, We, att, bias, H, C):
    n = x.shape[0]
    xl = (x @ Wl + bl).reshape(n, H, C)
    xr = (x @ Wr + br).reshape(n, H, C)
    ee = (ea @ We).reshape(-1, H, C)
    e = jax.nn.leaky_relu(xl[src] + xr[dst] + ee, negative_slope=0.2)
    alpha = jnp.einsum('ehc,hc->eh', e, att)
    m = jax.ops.segment_max(alpha, dst, num_segments=n)
    m = jax.lax.stop_gradient(jnp.where(jnp.isfinite(m), m, 0.0))
    ex = jnp.exp(alpha - m[dst])
    den = jax.ops.segment_sum(ex, dst, num_segments=n)
    a = ex / (den[dst] + 1e-16)
    out = jax.ops.segment_sum(a[:, :, None] * xl[src], dst, num_segments=n)
    return out.reshape(n, H * C) + bias


def kernel(x, edge_index, edge_attr, batch, Wl1, bl1, Wr1, br1, We1, att1,
           bias1, Wl2, bl2, Wr2, br2, We2, att2, bias2, Wlin, blin):
    n = x.shape[0]
    loop = jnp.arange(n, dtype=edge_index.dtype)
    mean_ea = jnp.mean(edge_attr, axis=0)
    src = jnp.concatenate([edge_index[0], loop])
    dst = jnp.concatenate([edge_index[1], loop])
    ea = jnp.concatenate(
        [edge_attr, jnp.tile(mean_ea[None, :], (n, 1))], axis=0)
    h = _gatv2_ref(x, src, dst, ea, Wl1, bl1, Wr1, br1, We1, att1, bias1, 2, 64)
    h = jax.nn.relu(h)
    h = _gatv2_ref(x=h, src=src, dst=dst, ea=ea, Wl=Wl2, bl=bl2, Wr=Wr2,
                   br=br2, We=We2, att=att2, bias=bias2, H=1, C=128)
    npad = 50000
    return _pool_final(h, batch, Wlin, blin)


# speedup vs baseline: 6.6793x; 6.6654x over previous
import functools

import jax
import jax.numpy as jnp
from jax import lax
from jax.experimental import pallas as pl
from jax.experimental.pallas import tpu as pltpu
from jax.experimental.pallas import tpu_sc as plsc

_N = 50000
_E = 800000
_E2 = _E + _N
_G = 64

_NC, _NS = 2, 16
_NW = _NC * _NS
_NPT = 1568
_NPAD = _NW * _NPT
_CH = 256
_EPT = 26624
_EPAD = _NW * _EPT
_E1PAD = 800256
_RPS = 1584


def _mean_body(ea_ref, out_ref, acc_ref):
    i = pl.program_id(0)

    @pl.when(i == 0)
    def _():
        acc_ref[...] = jnp.zeros_like(acc_ref)

    acc_ref[...] += ea_ref[...]

    @pl.when(i == pl.num_programs(0) - 1)
    def _():
        out_ref[...] = jnp.sum(acc_ref[...], axis=0, keepdims=True) / _E


def _mean_rows(ea):
    blk = 256
    return pl.pallas_call(
        _mean_body,
        grid=(_E // blk,),
        in_specs=[pl.BlockSpec((blk, 16), lambda i: (i, 0))],
        out_specs=pl.BlockSpec((1, 16), lambda i: (0, 0)),
        out_shape=jax.ShapeDtypeStruct((1, 16), jnp.float32),
        scratch_shapes=[pltpu.VMEM((blk, 16), jnp.float32)],
    )(ea)


def _ee_body(ea_ref, w1_ref, w2_ref, o1_ref, o2_ref):
    ea = ea_ref[...]
    o1_ref[...] = jnp.dot(ea, w1_ref[...], preferred_element_type=jnp.float32)
    o2_ref[...] = jnp.dot(ea, w2_ref[...], preferred_element_type=jnp.float32)


def _edge_embed(ea_ext, We1, We2):
    blk = 512
    return pl.pallas_call(
        _ee_body,
        grid=(_E1PAD // blk,),
        in_specs=[
            pl.BlockSpec((blk, 16), lambda i: (i, 0)),
            pl.BlockSpec((16, 128), lambda i: (0, 0)),
            pl.BlockSpec((16, 128), lambda i: (0, 0)),
        ],
        out_specs=[
            pl.BlockSpec((blk, 128), lambda i: (i, 0)),
            pl.BlockSpec((blk, 128), lambda i: (i, 0)),
        ],
        out_shape=[
            jax.ShapeDtypeStruct((_E1PAD, 128), jnp.float32),
            jax.ShapeDtypeStruct((_E1PAD, 128), jnp.float32),
        ],
    )(ea_ext, We1, We2)


def _proj_body(x_ref, wl_ref, bl_ref, wr_ref, br_ref, xl_ref, xr_ref):
    x = x_ref[...]
    xl_ref[...] = jnp.dot(x, wl_ref[...],
                          preferred_element_type=jnp.float32) + bl_ref[...]
    xr_ref[...] = jnp.dot(x, wr_ref[...],
                          preferred_element_type=jnp.float32) + br_ref[...]


def _proj(x, Wl, bl, Wr, br):
    blk = 512
    din = x.shape[1]
    return pl.pallas_call(
        _proj_body,
        grid=(_NPAD // blk,),
        in_specs=[
            pl.BlockSpec((blk, din), lambda i: (i, 0)),
            pl.BlockSpec((din, 128), lambda i: (0, 0)),
            pl.BlockSpec((1, 128), lambda i: (0, 0)),
            pl.BlockSpec((din, 128), lambda i: (0, 0)),
            pl.BlockSpec((1, 128), lambda i: (0, 0)),
        ],
        out_specs=[
            pl.BlockSpec((blk, 128), lambda i: (i, 0)),
            pl.BlockSpec((blk, 128), lambda i: (i, 0)),
        ],
        out_shape=[
            jax.ShapeDtypeStruct((_NPAD, 128), jnp.float32),
            jax.ShapeDtypeStruct((_NPAD, 128), jnp.float32),
        ],
    )(x, Wl, bl.reshape(1, 128), Wr, br.reshape(1, 128))


def _pool_body(h_ref, batch_ref, wlin_ref, blin_ref, out_ref, acc_ref,
               cnt_ref):
    i = pl.program_id(0)

    @pl.when(i == 0)
    def _():
        acc_ref[...] = jnp.zeros_like(acc_ref)
        cnt_ref[...] = jnp.zeros_like(cnt_ref)

    blk = h_ref.shape[0]
    bb = batch_ref[...].reshape(1, blk)
    onehot = (lax.broadcasted_iota(jnp.int32, (_G, blk), 0) ==
              jnp.broadcast_to(bb, (_G, blk))).astype(jnp.float32)
    acc_ref[...] += jnp.dot(onehot, h_ref[...],
                            preferred_element_type=jnp.float32)
    cnt_ref[...] += jnp.sum(onehot, axis=1, keepdims=True)

    @pl.when(i == pl.num_programs(0) - 1)
    def _():
        pooled = acc_ref[...] / jnp.maximum(cnt_ref[...], 1.0)
        out = jnp.dot(pooled, wlin_ref[...],
                      preferred_element_type=jnp.float32) + blin_ref[...]
        out_ref[...] = jax.nn.sigmoid(out)


def _pool_final(h, batchp, Wlin, blin):
    blk = 512
    grid = _NPAD // blk
    return pl.pallas_call(
        _pool_body,
        grid=(grid,),
        in_specs=[
            pl.BlockSpec((blk, 128), lambda i: (i, 0)),
            pl.BlockSpec((1, 1, blk), lambda i: (i, 0, 0)),
            pl.BlockSpec((128, 1), lambda i: (0, 0)),
            pl.BlockSpec((1, 1), lambda i: (0, 0)),
        ],
        out_specs=pl.BlockSpec((_G, 1), lambda i: (0, 0)),
        out_shape=jax.ShapeDtypeStruct((_G, 1), jnp.float32),
        scratch_shapes=[
            pltpu.VMEM((_G, 128), jnp.float32),
            pltpu.VMEM((_G, 1), jnp.float32),
        ],
    )(h, batchp.reshape(grid, 1, blk), Wlin, blin.reshape(1, 1))


def _wid():
    return lax.axis_index("s") * _NC + lax.axis_index("c")


_GDN = lax.GatherDimensionNumbers(
    offset_dims=(), collapsed_slice_dims=(0,), start_index_map=(0,))


def _permute(v, idx):
    return lax.gather(v, idx[:, None], _GDN, (1,),
                      mode=lax.GatherScatterMode.PROMISE_IN_BOUNDS)


def _vsum(v, rots):
    for idx in rots:
        v = v + _permute(v, idx)
    return v[0]


def _make_rots():
    return [(jnp.arange(16, dtype=jnp.int32) + s) % 16 for s in (1, 2, 4, 8)]


def _alpha_kernel(H):
    mesh = plsc.VectorSubcoreMesh(core_axis_name="c", subcore_axis_name="s")

    def body(xl, xr, ee, srcs, dsts, permc, attf, alpha_out,
             srcv, dstv, permv, gxl, gxr, gee, attv, a0v, a1v, s1, s2, s3):
        w = _wid()
        lane0 = lax.iota(jnp.int32, 16) == 0
        rots = _make_rots()
        pltpu.sync_copy(attf, attv)
        att_blk = [attv[pl.ds(cc * 16, 16)] for cc in range(8)]

        def chunk(k, carry):
            base = w * _EPT + k * _CH
            pltpu.sync_copy(srcs.at[pl.ds(base, _CH)], srcv)
            pltpu.sync_copy(dsts.at[pl.ds(base, _CH)], dstv)
            pltpu.sync_copy(permc.at[pl.ds(base, _CH)], permv)
            c1 = pltpu.async_copy(xl.at[srcv], gxl, s1)
            c2 = pltpu.async_copy(xr.at[dstv], gxr, s2)
            c3 = pltpu.async_copy(ee.at[permv], gee, s3)
            c1.wait()
            c2.wait()
            c3.wait()

            def edge(e, carry2):
                vs0 = jnp.zeros((16,), jnp.float32)
                vs1 = jnp.zeros((16,), jnp.float32)
                for cc in range(8):
                    sl = pl.ds(cc * 16, 16)
                    v = gxl[e, sl] + gxr[e, sl] + gee[e, sl]
                    v = jnp.where(v > 0, v, 0.2 * v)
                    av = v * att_blk[cc]
                    if H == 2 and cc >= 4:
                        vs1 = vs1 + av
                    else:
                        vs0 = vs0 + av
                es = jnp.full((16,), e, jnp.int32)
                plsc.store_scatter(a0v, [es],
                                   jnp.full((16,), _vsum(vs0, rots)),
                                   mask=lane0)
                if H == 2:
                    plsc.store_scatter(a1v, [es],
                                       jnp.full((16,), _vsum(vs1, rots)),
                                       mask=lane0)
                return carry2

            lax.fori_loop(0, _CH, edge, 0)
            pltpu.sync_copy(a0v, alpha_out.at[0, pl.ds(base, _CH)])
            if H == 2:
                pltpu.sync_copy(a1v, alpha_out.at[1, pl.ds(base, _CH)])
            return carry

        lax.fori_loop(0, _EPT // _CH, chunk, 0)

    return pl.kernel(
        body,
        out_type=jax.ShapeDtypeStruct((H, _EPAD), jnp.float32),
        mesh=mesh,
        compiler_params=pltpu.CompilerParams(needs_layout_passes=False),
        scratch_types=[
            pltpu.VMEM((_CH,), jnp.int32),
            pltpu.VMEM((_CH,), jnp.int32),
            pltpu.VMEM((_CH,), jnp.int32),
            pltpu.VMEM((_CH, 128), jnp.float32),
            pltpu.VMEM((_CH, 128), jnp.float32),
            pltpu.VMEM((_CH, 128), jnp.float32),
            pltpu.VMEM((128,), jnp.float32),
            pltpu.VMEM((_CH,), jnp.float32),
            pltpu.VMEM((_CH,), jnp.float32),
            pltpu.SemaphoreType.DMA,
            pltpu.SemaphoreType.DMA,
            pltpu.SemaphoreType.DMA,
        ],
    )


def _agg_kernel(H, relu):
    mesh = plsc.VectorSubcoreMesh(core_axis_name="c", subcore_axis_name="s")

    def body(xl, alpha, srcs, dsts, rowptr, biasf, out,
             rpv, m0, m1, d0, d1, srcv, dstv, a0v, a1v, w0v, w1v, gxl,
             accv, stg, biasv, s1):
        w = _wid()
        n0 = w * _NPT
        lane0 = lax.iota(jnp.int32, 16) == 0
        pltpu.sync_copy(rowptr.at[pl.ds(n0, _RPS)], rpv)
        pltpu.sync_copy(biasf, biasv)

        def init(i, carry):
            m0[pl.ds(i * 16, 16)] = jnp.full((16,), -1e30, jnp.float32)
            m1[pl.ds(i * 16, 16)] = jnp.full((16,), -1e30, jnp.float32)
            d0[pl.ds(i * 16, 16)] = jnp.zeros((16,), jnp.float32)
            d1[pl.ds(i * 16, 16)] = jnp.zeros((16,), jnp.float32)
            return carry

        lax.fori_loop(0, (_NPT + 16) // 16, init, 0)
        for cc in range(8):
            accv[pl.ds(cc * 16, 16)] = jnp.zeros((16,), jnp.float32)

        def zrow(e, carry):
            for cc in range(8):
                gxl[e, pl.ds(cc * 16, 16)] = jnp.zeros((16,), jnp.float32)
            return carry

        lax.fori_loop(0, _CH, zrow, 0)

        def zfill(i, carry):
            pltpu.sync_copy(gxl, out.at[pl.ds(n0 + i * _CH, _CH), :])
            return carry

        lax.fori_loop(0, _NPT // _CH, zfill, 0)
        pltpu.sync_copy(gxl.at[pl.ds(0, _NPT % _CH), :],
                        out.at[pl.ds(n0 + (_NPT // _CH) * _CH,
                                     _NPT % _CH), :])

        rp0 = rpv[pl.ds(0, 16)][0]
        rp1 = rpv[pl.ds(_NPT, 16)][0]
        kstart = rp0 // _CH
        kend = (rp1 + _CH - 1) // _CH

        def stage(k, also_src):
            cb = k * _CH
            pltpu.sync_copy(dsts.at[pl.ds(cb, _CH)], dstv.at[pl.ds(0, _CH)])
            pltpu.sync_copy(alpha.at[0, pl.ds(cb, _CH)], a0v.at[pl.ds(0, _CH)])
            if H == 2:
                pltpu.sync_copy(alpha.at[1, pl.ds(cb, _CH)],
                                a1v.at[pl.ds(0, _CH)])
            if also_src:
                pltpu.sync_copy(srcs.at[pl.ds(cb, _CH)], srcv)
            lo = jnp.maximum(cb, rp0) - cb
            hi = jnp.minimum(cb + _CH, rp1) - cb
            return lo, hi

        def sweep_a(k, carry):
            lo, hi = stage(k, False)

            def per_edge(i, c2):
                nl = dstv[pl.ds(i, 16)][0] - n0
                nls = jnp.full((16,), nl, jnp.int32)
                av = a0v[pl.ds(i, 16)]
                mo = m0[pl.ds(nl, 16)]
                plsc.store_scatter(m0, [nls], jnp.maximum(mo, av), mask=lane0)
                if H == 2:
                    av1 = a1v[pl.ds(i, 16)]
                    mo1 = m1[pl.ds(nl, 16)]
                    plsc.store_scatter(m1, [nls], jnp.maximum(mo1, av1),
                                       mask=lane0)
                return c2

            lax.fori_loop(lo, hi, per_edge, 0)
            return carry

        lax.fori_loop(kstart, kend, sweep_a, 0)

        def sweep_b(k, carry):
            lo, hi = stage(k, False)

            def vec(g, c2):
                sl = pl.ds(g * 16, 16)
                nl = jnp.clip(dstv[sl] - n0, 0, _NPT - 1)
                w0v[sl] = jnp.exp(a0v[sl] - plsc.load_gather(m0, [nl]))
                if H == 2:
                    w1v[sl] = jnp.exp(a1v[sl] - plsc.load_gather(m1, [nl]))
                return c2

            lax.fori_loop(0, _CH // 16, vec, 0)

            def per_edge(i, c2):
                nl = dstv[pl.ds(i, 16)][0] - n0
                nls = jnp.full((16,), nl, jnp.int32)
                dv = d0[pl.ds(nl, 16)]
                plsc.store_scatter(d0, [nls], dv + w0v[pl.ds(i, 16)],
                                   mask=lane0)
                if H == 2:
                    dv1 = d1[pl.ds(nl, 16)]
                    plsc.store_scatter(d1, [nls], dv1 + w1v[pl.ds(i, 16)],
                                       mask=lane0)
                return c2

            lax.fori_loop(lo, hi, per_edge, 0)
            return carry

        lax.fori_loop(kstart, kend, sweep_b, 0)

        def flush(cur):
            for cc in range(8):
                sl = pl.ds(cc * 16, 16)
                v = accv[sl] + biasv[sl]
                if relu:
                    v = jnp.maximum(v, 0.0)
                stg[sl] = v
                accv[sl] = jnp.zeros((16,), jnp.float32)
            pltpu.sync_copy(stg, out.at[cur])

        def sweep_c(k, cur):
            lo, hi = stage(k, True)
            cp = pltpu.async_copy(xl.at[srcv], gxl, s1)

            def vec(g, c2):
                sl = pl.ds(g * 16, 16)
                nl = jnp.clip(dstv[sl] - n0, 0, _NPT - 1)
                ex0 = jnp.exp(a0v[sl] - plsc.load_gather(m0, [nl]))
                w0v[sl] = ex0 / (plsc.load_gather(d0, [nl]) + 1e-16)
                if H == 2:
                    ex1 = jnp.exp(a1v[sl] - plsc.load_gather(m1, [nl]))
                    w1v[sl] = ex1 / (plsc.load_gather(d1, [nl]) + 1e-16)
                return c2

            lax.fori_loop(0, _CH // 16, vec, 0)
            cp.wait()

            def per_edge(i, cur2):
                nd = dstv[pl.ds(i, 16)][0]
                changed = nd != cur2

                @pl.when(changed & (cur2 >= 0))
                def _():
                    flush(cur2)

                w0s = w0v[pl.ds(i, 16)][0]
                if H == 2:
                    w1s = w1v[pl.ds(i, 16)][0]
                for cc in range(8):
                    ws = w1s if (H == 2 and cc >= 4) else w0s
                    sl = pl.ds(cc * 16, 16)
                    accv[sl] = accv[sl] + ws * gxl[i, sl]
                return nd

            return lax.fori_loop(lo, hi, per_edge, cur)

        cur = lax.fori_loop(kstart, kend, sweep_c, jnp.int32(-1))

        @pl.when(cur >= 0)
        def _():
            flush(cur)

    return pl.kernel(
        body,
        out_type=jax.ShapeDtypeStruct((_NPAD, 128), jnp.float32),
        mesh=mesh,
        compiler_params=pltpu.CompilerParams(needs_layout_passes=False),
        scratch_types=[
            pltpu.VMEM((_RPS,), jnp.int32),
            pltpu.VMEM((_NPT + 16,), jnp.float32),
            pltpu.VMEM((_NPT + 16,), jnp.float32),
            pltpu.VMEM((_NPT + 16,), jnp.float32),
            pltpu.VMEM((_NPT + 16,), jnp.float32),
            pltpu.VMEM((_CH,), jnp.int32),
            pltpu.VMEM((_CH + 16,), jnp.int32),
            pltpu.VMEM((_CH + 16,), jnp.float32),
            pltpu.VMEM((_CH + 16,), jnp.float32),
            pltpu.VMEM((_CH + 16,), jnp.float32),
            pltpu.VMEM((_CH + 16,), jnp.float32),
            pltpu.VMEM((_CH, 128), jnp.float32),
            pltpu.VMEM((128,), jnp.float32),
            pltpu.VMEM((128,), jnp.float32),
            pltpu.VMEM((128,), jnp.float32),
            pltpu.SemaphoreType.DMA,
        ],
    )


def kernel(x, edge_index, edge_attr, batch, Wl1, bl1, Wr1, br1, We1, att1,
           bias1, Wl2, bl2, Wr2, br2, We2, att2, bias2, Wlin, blin):
    loop = jnp.arange(_N, dtype=jnp.int32)
    src2 = jnp.concatenate([edge_index[0], loop])
    dst2 = jnp.concatenate([edge_index[1], loop])
    perm = jnp.argsort(dst2).astype(jnp.int32)
    dst_s = dst2[perm]
    src_s = src2[perm]
    permc = jnp.minimum(perm, _E)

    srcs = jnp.zeros((_EPAD,), jnp.int32).at[:_E2].set(src_s)
    dsts = jnp.full((_EPAD,), _NPAD, jnp.int32).at[:_E2].set(dst_s)
    dstg = jnp.minimum(dsts, _NPAD - 1)
    permcp = jnp.zeros((_EPAD,), jnp.int32).at[:_E2].set(permc)
    rowptr = jnp.searchsorted(dsts, jnp.arange(_NPAD + 1, dtype=jnp.int32)
                              ).astype(jnp.int32)
    rowptr = jnp.concatenate(
        [rowptr, jnp.full((_NW * _NPT + _RPS - _NPAD - 1,), _E2, jnp.int32)])

    mean_row = _mean_rows(edge_attr)
    ea_ext = jnp.concatenate(
        [edge_attr, jnp.broadcast_to(mean_row, (_E1PAD - _E, 16))])
    ee1, ee2 = _edge_embed(ea_ext, We1, We2)

    xpad = jnp.zeros((_NPAD, x.shape[1]), jnp.float32).at[:_N].set(x)
    xl1, xr1 = _proj(xpad, Wl1, bl1, Wr1, br1)

    alpha1 = _alpha_kernel(2)(xl1, xr1, ee1, srcs, dstg, permcp,
                              att1.reshape(128))
    h = _agg_kernel(2, True)(xl1, alpha1, srcs, dsts, rowptr,
                             bias1.reshape(128))

    xl2, xr2 = _proj(h, Wl2, bl2, Wr2, br2)
    alpha2 = _alpha_kernel(1)(xl2, xr2, ee2, srcs, dstg, permcp,
                              att2.reshape(128))
    h2 = _agg_kernel(1, False)(xl2, alpha2, srcs, dsts, rowptr,
                               bias2.reshape(128))

    batchp = jnp.full((_NPAD,), _G, jnp.int32).at[:_N].set(batch)
    return _pool_final(h2, batchp, Wlin, blin)


# speedup vs baseline: 7.7353x; 1.1581x over previous
import functools

import jax
import jax.numpy as jnp
from jax import lax
from jax.experimental import pallas as pl
from jax.experimental.pallas import tpu as pltpu
from jax.experimental.pallas import tpu_sc as plsc

_N = 50000
_E = 800000
_E2 = _E + _N
_G = 64

_NC, _NS = 2, 16
_NW = _NC * _NS
_NPT = 1568
_NPAD = _NW * _NPT
_CH = 256
_EPT = 26624
_EPAD = _NW * _EPT
_E1PAD = 800256
_RPS = 1584


def _mean_body(ea_ref, out_ref, acc_ref):
    i = pl.program_id(0)

    @pl.when(i == 0)
    def _():
        acc_ref[...] = jnp.zeros_like(acc_ref)

    acc_ref[...] += ea_ref[...]

    @pl.when(i == pl.num_programs(0) - 1)
    def _():
        out_ref[...] = jnp.sum(acc_ref[...], axis=0, keepdims=True) / _E


def _mean_rows(ea):
    blk = 256
    return pl.pallas_call(
        _mean_body,
        grid=(_E // blk,),
        in_specs=[pl.BlockSpec((blk, 16), lambda i: (i, 0))],
        out_specs=pl.BlockSpec((1, 16), lambda i: (0, 0)),
        out_shape=jax.ShapeDtypeStruct((1, 16), jnp.float32),
        scratch_shapes=[pltpu.VMEM((blk, 16), jnp.float32)],
    )(ea)


def _ee_body(ea_ref, w1_ref, w2_ref, o1_ref, o2_ref):
    ea = ea_ref[...]
    o1_ref[...] = jnp.dot(ea, w1_ref[...], preferred_element_type=jnp.float32)
    o2_ref[...] = jnp.dot(ea, w2_ref[...], preferred_element_type=jnp.float32)


def _edge_embed(ea_ext, We1, We2):
    blk = 512
    return pl.pallas_call(
        _ee_body,
        grid=(_E1PAD // blk,),
        in_specs=[
            pl.BlockSpec((blk, 16), lambda i: (i, 0)),
            pl.BlockSpec((16, 128), lambda i: (0, 0)),
            pl.BlockSpec((16, 128), lambda i: (0, 0)),
        ],
        out_specs=[
            pl.BlockSpec((blk, 128), lambda i: (i, 0)),
            pl.BlockSpec((blk, 128), lambda i: (i, 0)),
        ],
        out_shape=[
            jax.ShapeDtypeStruct((_E1PAD, 128), jnp.float32),
            jax.ShapeDtypeStruct((_E1PAD, 128), jnp.float32),
        ],
    )(ea_ext, We1, We2)


def _proj_body(x_ref, wl_ref, bl_ref, wr_ref, br_ref, xl_ref, xr_ref):
    x = x_ref[...]
    xl_ref[...] = jnp.dot(x, wl_ref[...],
                          preferred_element_type=jnp.float32) + bl_ref[...]
    xr_ref[...] = jnp.dot(x, wr_ref[...],
                          preferred_element_type=jnp.float32) + br_ref[...]


def _proj(x, Wl, bl, Wr, br):
    blk = 512
    din = x.shape[1]
    return pl.pallas_call(
        _proj_body,
        grid=(_NPAD // blk,),
        in_specs=[
            pl.BlockSpec((blk, din), lambda i: (i, 0)),
            pl.BlockSpec((din, 128), lambda i: (0, 0)),
            pl.BlockSpec((1, 128), lambda i: (0, 0)),
            pl.BlockSpec((din, 128), lambda i: (0, 0)),
            pl.BlockSpec((1, 128), lambda i: (0, 0)),
        ],
        out_specs=[
            pl.BlockSpec((blk, 128), lambda i: (i, 0)),
            pl.BlockSpec((blk, 128), lambda i: (i, 0)),
        ],
        out_shape=[
            jax.ShapeDtypeStruct((_NPAD, 128), jnp.float32),
            jax.ShapeDtypeStruct((_NPAD, 128), jnp.float32),
        ],
    )(x, Wl, bl.reshape(1, 128), Wr, br.reshape(1, 128))


def _pool_body(h_ref, batch_ref, wlin_ref, blin_ref, out_ref, acc_ref,
               cnt_ref):
    i = pl.program_id(0)

    @pl.when(i == 0)
    def _():
        acc_ref[...] = jnp.zeros_like(acc_ref)
        cnt_ref[...] = jnp.zeros_like(cnt_ref)

    blk = h_ref.shape[0]
    bb = batch_ref[...].reshape(1, blk)
    onehot = (lax.broadcasted_iota(jnp.int32, (_G, blk), 0) ==
              jnp.broadcast_to(bb, (_G, blk))).astype(jnp.float32)
    acc_ref[...] += jnp.dot(onehot, h_ref[...],
                            preferred_element_type=jnp.float32)
    cnt_ref[...] += jnp.sum(onehot, axis=1, keepdims=True)

    @pl.when(i == pl.num_programs(0) - 1)
    def _():
        pooled = acc_ref[...] / jnp.maximum(cnt_ref[...], 1.0)
        out = jnp.dot(pooled, wlin_ref[...],
                      preferred_element_type=jnp.float32) + blin_ref[...]
        out_ref[...] = jax.nn.sigmoid(out)


def _pool_final(h, batchp, Wlin, blin):
    blk = 512
    grid = _NPAD // blk
    return pl.pallas_call(
        _pool_body,
        grid=(grid,),
        in_specs=[
            pl.BlockSpec((blk, 128), lambda i: (i, 0)),
            pl.BlockSpec((1, 1, blk), lambda i: (i, 0, 0)),
            pl.BlockSpec((128, 1), lambda i: (0, 0)),
            pl.BlockSpec((1, 1), lambda i: (0, 0)),
        ],
        out_specs=pl.BlockSpec((_G, 1), lambda i: (0, 0)),
        out_shape=jax.ShapeDtypeStruct((_G, 1), jnp.float32),
        scratch_shapes=[
            pltpu.VMEM((_G, 128), jnp.float32),
            pltpu.VMEM((_G, 1), jnp.float32),
        ],
    )(h, batchp.reshape(grid, 1, blk), Wlin, blin.reshape(1, 1))


def _wid():
    return lax.axis_index("s") * _NC + lax.axis_index("c")


_GDN = lax.GatherDimensionNumbers(
    offset_dims=(), collapsed_slice_dims=(0,), start_index_map=(0,))


def _permute(v, idx):
    return lax.gather(v, idx[:, None], _GDN, (1,),
                      mode=lax.GatherScatterMode.PROMISE_IN_BOUNDS)


def _vsum(v, rots):
    for idx in rots:
        v = v + _permute(v, idx)
    return v[0]


def _make_rots():
    return [(jnp.arange(16, dtype=jnp.int32) + s) % 16 for s in (1, 2, 4, 8)]


def _alpha_kernel(H):
    mesh = plsc.VectorSubcoreMesh(core_axis_name="c", subcore_axis_name="s")

    def body(xl, xr, ee, srcs, dsts, permc, attf, alpha_out,
             srcv, dstv, permv, gxl, gxr, gee, attv, a0v, a1v, s1, s2, s3):
        w = _wid()
        lane0 = lax.iota(jnp.int32, 16) == 0
        rots = _make_rots()
        pltpu.sync_copy(attf, attv)
        att_blk = [attv[pl.ds(cc * 16, 16)] for cc in range(8)]

        def chunk(k, carry):
            base = w * _EPT + k * _CH
            pltpu.sync_copy(srcs.at[pl.ds(base, _CH)], srcv)
            pltpu.sync_copy(dsts.at[pl.ds(base, _CH)], dstv)
            pltpu.sync_copy(permc.at[pl.ds(base, _CH)], permv)
            c1 = pltpu.async_copy(xl.at[srcv], gxl, s1)
            c2 = pltpu.async_copy(xr.at[dstv], gxr, s2)
            c3 = pltpu.async_copy(ee.at[permv], gee, s3)
            c1.wait()
            c2.wait()
            c3.wait()

            def edge(e, carry2):
                vs0 = jnp.zeros((16,), jnp.float32)
                vs1 = jnp.zeros((16,), jnp.float32)
                for cc in range(8):
                    sl = pl.ds(cc * 16, 16)
                    v = gxl[e, sl] + gxr[e, sl] + gee[e, sl]
                    v = jnp.where(v > 0, v, 0.2 * v)
                    av = v * att_blk[cc]
                    if H == 2 and cc >= 4:
                        vs1 = vs1 + av
                    else:
                        vs0 = vs0 + av
                es = jnp.full((16,), e, jnp.int32)
                plsc.store_scatter(a0v, [es],
                                   jnp.full((16,), _vsum(vs0, rots)),
                                   mask=lane0)
                if H == 2:
                    plsc.store_scatter(a1v, [es],
                                       jnp.full((16,), _vsum(vs1, rots)),
                                       mask=lane0)
                return carry2

            lax.fori_loop(0, _CH, edge, 0, unroll=2)
            pltpu.sync_copy(a0v, alpha_out.at[0, pl.ds(base, _CH)])
            if H == 2:
                pltpu.sync_copy(a1v, alpha_out.at[1, pl.ds(base, _CH)])
            return carry

        lax.fori_loop(0, _EPT // _CH, chunk, 0)

    return pl.kernel(
        body,
        out_type=jax.ShapeDtypeStruct((H, _EPAD), jnp.float32),
        mesh=mesh,
        compiler_params=pltpu.CompilerParams(needs_layout_passes=False),
        scratch_types=[
            pltpu.VMEM((_CH,), jnp.int32),
            pltpu.VMEM((_CH,), jnp.int32),
            pltpu.VMEM((_CH,), jnp.int32),
            pltpu.VMEM((_CH, 128), jnp.float32),
            pltpu.VMEM((_CH, 128), jnp.float32),
            pltpu.VMEM((_CH, 128), jnp.float32),
            pltpu.VMEM((128,), jnp.float32),
            pltpu.VMEM((_CH,), jnp.float32),
            pltpu.VMEM((_CH,), jnp.float32),
            pltpu.SemaphoreType.DMA,
            pltpu.SemaphoreType.DMA,
            pltpu.SemaphoreType.DMA,
        ],
    )


def _agg_kernel(H, relu):
    mesh = plsc.VectorSubcoreMesh(core_axis_name="c", subcore_axis_name="s")

    def body(xl, alpha, srcs, dsts, rowptr, biasf, out,
             rpv, m0, m1, d0, d1, srcv, dstv, a0v, a1v, w0v, w1v, gxl,
             accv, stg, biasv, s1):
        w = _wid()
        n0 = w * _NPT
        lane0 = lax.iota(jnp.int32, 16) == 0
        pltpu.sync_copy(rowptr.at[pl.ds(n0, _RPS)], rpv)
        pltpu.sync_copy(biasf, biasv)

        def init(i, carry):
            m0[pl.ds(i * 16, 16)] = jnp.full((16,), -1e30, jnp.float32)
            m1[pl.ds(i * 16, 16)] = jnp.full((16,), -1e30, jnp.float32)
            d0[pl.ds(i * 16, 16)] = jnp.zeros((16,), jnp.float32)
            d1[pl.ds(i * 16, 16)] = jnp.zeros((16,), jnp.float32)
            return carry

        lax.fori_loop(0, (_NPT + 16) // 16, init, 0)

        def zrow(e, carry):
            for cc in range(8):
                gxl[e, pl.ds(cc * 16, 16)] = jnp.zeros((16,), jnp.float32)
            return carry

        lax.fori_loop(0, _CH, zrow, 0)

        def zfill(i, carry):
            pltpu.sync_copy(gxl, out.at[pl.ds(n0 + i * _CH, _CH), :])
            return carry

        lax.fori_loop(0, _NPT // _CH, zfill, 0)
        pltpu.sync_copy(gxl.at[pl.ds(0, _NPT % _CH), :],
                        out.at[pl.ds(n0 + (_NPT // _CH) * _CH,
                                     _NPT % _CH), :])

        rp0 = rpv[pl.ds(0, 16)][0]
        rp1 = rpv[pl.ds(_NPT, 16)][0]
        kstart = rp0 // _CH
        kend = (rp1 + _CH - 1) // _CH

        def stage(k, also_src):
            cb = k * _CH
            pltpu.sync_copy(dsts.at[pl.ds(cb, _CH)], dstv.at[pl.ds(0, _CH)])
            pltpu.sync_copy(alpha.at[0, pl.ds(cb, _CH)], a0v.at[pl.ds(0, _CH)])
            if H == 2:
                pltpu.sync_copy(alpha.at[1, pl.ds(cb, _CH)],
                                a1v.at[pl.ds(0, _CH)])
            if also_src:
                pltpu.sync_copy(srcs.at[pl.ds(cb, _CH)], srcv)
            lo = jnp.maximum(cb, rp0) - cb
            hi = jnp.minimum(cb + _CH, rp1) - cb
            return lo, hi

        def sweep_a(k, carry):
            lo, hi = stage(k, False)

            def per_edge(i, c2):
                nl = dstv[pl.ds(i, 16)][0] - n0
                nls = jnp.full((16,), nl, jnp.int32)
                av = a0v[pl.ds(i, 16)]
                mo = m0[pl.ds(nl, 16)]
                plsc.store_scatter(m0, [nls], jnp.maximum(mo, av), mask=lane0)
                if H == 2:
                    av1 = a1v[pl.ds(i, 16)]
                    mo1 = m1[pl.ds(nl, 16)]
                    plsc.store_scatter(m1, [nls], jnp.maximum(mo1, av1),
                                       mask=lane0)
                return c2

            lax.fori_loop(lo, hi, per_edge, 0)
            return carry

        lax.fori_loop(kstart, kend, sweep_a, 0)

        def sweep_b(k, carry):
            lo, hi = stage(k, False)

            def vec(g, c2):
                sl = pl.ds(g * 16, 16)
                nl = jnp.clip(dstv[sl] - n0, 0, _NPT - 1)
                w0v[sl] = jnp.exp(a0v[sl] - plsc.load_gather(m0, [nl]))
                if H == 2:
                    w1v[sl] = jnp.exp(a1v[sl] - plsc.load_gather(m1, [nl]))
                return c2

            lax.fori_loop(0, _CH // 16, vec, 0)

            def per_edge(i, c2):
                nl = dstv[pl.ds(i, 16)][0] - n0
                nls = jnp.full((16,), nl, jnp.int32)
                dv = d0[pl.ds(nl, 16)]
                plsc.store_scatter(d0, [nls], dv + w0v[pl.ds(i, 16)],
                                   mask=lane0)
                if H == 2:
                    dv1 = d1[pl.ds(nl, 16)]
                    plsc.store_scatter(d1, [nls], dv1 + w1v[pl.ds(i, 16)],
                                       mask=lane0)
                return c2

            lax.fori_loop(lo, hi, per_edge, 0)
            return carry

        lax.fori_loop(kstart, kend, sweep_b, 0)

        zero8 = tuple(jnp.zeros((16,), jnp.float32) for _ in range(8))

        def flush(cur, acc):
            for cc in range(8):
                sl = pl.ds(cc * 16, 16)
                v = acc[cc] + biasv[sl]
                if relu:
                    v = jnp.maximum(v, 0.0)
                stg[sl] = v
            pltpu.sync_copy(stg, out.at[cur])

        def sweep_c(k, carry):
            lo, hi = stage(k, True)
            cp = pltpu.async_copy(xl.at[srcv], gxl, s1)

            def vec(g, c2):
                sl = pl.ds(g * 16, 16)
                nl = jnp.clip(dstv[sl] - n0, 0, _NPT - 1)
                ex0 = jnp.exp(a0v[sl] - plsc.load_gather(m0, [nl]))
                w0v[sl] = ex0 / (plsc.load_gather(d0, [nl]) + 1e-16)
                if H == 2:
                    ex1 = jnp.exp(a1v[sl] - plsc.load_gather(m1, [nl]))
                    w1v[sl] = ex1 / (plsc.load_gather(d1, [nl]) + 1e-16)
                return c2

            lax.fori_loop(0, _CH // 16, vec, 0)
            cp.wait()

            def per_edge(i, carry2):
                cur2 = carry2[0]
                acc = carry2[1:]
                nd = dstv[pl.ds(i, 16)][0]
                changed = nd != cur2

                @pl.when(changed & (cur2 >= 0))
                def _():
                    flush(cur2, acc)

                w0s = w0v[pl.ds(i, 16)][0]
                if H == 2:
                    w1s = w1v[pl.ds(i, 16)][0]
                nacc = []
                for cc in range(8):
                    ws = w1s if (H == 2 and cc >= 4) else w0s
                    a = jnp.where(changed, 0.0, acc[cc])
                    nacc.append(a + ws * gxl[i, pl.ds(cc * 16, 16)])
                return (nd,) + tuple(nacc)

            return lax.fori_loop(lo, hi, per_edge, carry)

        carry = lax.fori_loop(kstart, kend, sweep_c,
                              (jnp.int32(-1),) + zero8)

        @pl.when(carry[0] >= 0)
        def _():
            flush(carry[0], carry[1:])

    return pl.kernel(
        body,
        out_type=jax.ShapeDtypeStruct((_NPAD, 128), jnp.float32),
        mesh=mesh,
        compiler_params=pltpu.CompilerParams(needs_layout_passes=False),
        scratch_types=[
            pltpu.VMEM((_RPS,), jnp.int32),
            pltpu.VMEM((_NPT + 16,), jnp.float32),
            pltpu.VMEM((_NPT + 16,), jnp.float32),
            pltpu.VMEM((_NPT + 16,), jnp.float32),
            pltpu.VMEM((_NPT + 16,), jnp.float32),
            pltpu.VMEM((_CH,), jnp.int32),
            pltpu.VMEM((_CH + 16,), jnp.int32),
            pltpu.VMEM((_CH + 16,), jnp.float32),
            pltpu.VMEM((_CH + 16,), jnp.float32),
            pltpu.VMEM((_CH + 16,), jnp.float32),
            pltpu.VMEM((_CH + 16,), jnp.float32),
            pltpu.VMEM((_CH, 128), jnp.float32),
            pltpu.VMEM((128,), jnp.float32),
            pltpu.VMEM((128,), jnp.float32),
            pltpu.VMEM((128,), jnp.float32),
            pltpu.SemaphoreType.DMA,
        ],
    )


def kernel(x, edge_index, edge_attr, batch, Wl1, bl1, Wr1, br1, We1, att1,
           bias1, Wl2, bl2, Wr2, br2, We2, att2, bias2, Wlin, blin):
    loop = jnp.arange(_N, dtype=jnp.int32)
    src2 = jnp.concatenate([edge_index[0], loop])
    dst2 = jnp.concatenate([edge_index[1], loop])
    perm = jnp.argsort(dst2).astype(jnp.int32)
    dst_s = dst2[perm]
    src_s = src2[perm]
    permc = jnp.minimum(perm, _E)

    srcs = jnp.zeros((_EPAD,), jnp.int32).at[:_E2].set(src_s)
    dsts = jnp.full((_EPAD,), _NPAD, jnp.int32).at[:_E2].set(dst_s)
    dstg = jnp.minimum(dsts, _NPAD - 1)
    permcp = jnp.zeros((_EPAD,), jnp.int32).at[:_E2].set(permc)
    rowptr = jnp.searchsorted(dsts, jnp.arange(_NPAD + 1, dtype=jnp.int32)
                              ).astype(jnp.int32)
    rowptr = jnp.concatenate(
        [rowptr, jnp.full((_NW * _NPT + _RPS - _NPAD - 1,), _E2, jnp.int32)])

    mean_row = _mean_rows(edge_attr)
    ea_ext = jnp.concatenate(
        [edge_attr, jnp.broadcast_to(mean_row, (_E1PAD - _E, 16))])
    ee1, ee2 = _edge_embed(ea_ext, We1, We2)

    xpad = jnp.zeros((_NPAD, x.shape[1]), jnp.float32).at[:_N].set(x)
    xl1, xr1 = _proj(xpad, Wl1, bl1, Wr1, br1)

    alpha1 = _alpha_kernel(2)(xl1, xr1, ee1, srcs, dstg, permcp,
                              att1.reshape(128))
    h = _agg_kernel(2, True)(xl1, alpha1, srcs, dsts, rowptr,
                             bias1.reshape(128))

    xl2, xr2 = _proj(h, Wl2, bl2, Wr2, br2)
    alpha2 = _alpha_kernel(1)(xl2, xr2, ee2, srcs, dstg, permcp,
                              att2.reshape(128))
    h2 = _agg_kernel(1, False)(xl2, alpha2, srcs, dsts, rowptr,
                               bias2.reshape(128))

    batchp = jnp.full((_NPAD,), _G, jnp.int32).at[:_N].set(batch)
    return _pool_final(h2, batchp, Wlin, blin)


# speedup vs baseline: 7.7406x; 1.0007x over previous
import functools

import jax
import jax.numpy as jnp
from jax import lax
from jax.experimental import pallas as pl
from jax.experimental.pallas import tpu as pltpu
from jax.experimental.pallas import tpu_sc as plsc

_N = 50000
_E = 800000
_E2 = _E + _N
_G = 64

_NC, _NS = 2, 16
_NW = _NC * _NS
_NPT = 1568
_NPAD = _NW * _NPT
_CH = 256
_EPT = 26624
_EPAD = _NW * _EPT
_E1PAD = 800256
_RPS = 1584


def _mean_body(ea_ref, out_ref, acc_ref):
    i = pl.program_id(0)

    @pl.when(i == 0)
    def _():
        acc_ref[...] = jnp.zeros_like(acc_ref)

    acc_ref[...] += ea_ref[...]

    @pl.when(i == pl.num_programs(0) - 1)
    def _():
        out_ref[...] = jnp.sum(acc_ref[...], axis=0, keepdims=True) / _E


def _mean_rows(ea):
    blk = 256
    return pl.pallas_call(
        _mean_body,
        grid=(_E // blk,),
        in_specs=[pl.BlockSpec((blk, 16), lambda i: (i, 0))],
        out_specs=pl.BlockSpec((1, 16), lambda i: (0, 0)),
        out_shape=jax.ShapeDtypeStruct((1, 16), jnp.float32),
        scratch_shapes=[pltpu.VMEM((blk, 16), jnp.float32)],
    )(ea)


def _ee_body(ea_ref, w1_ref, w2_ref, o1_ref, o2_ref):
    ea = ea_ref[...]
    o1_ref[...] = jnp.dot(ea, w1_ref[...], preferred_element_type=jnp.float32)
    o2_ref[...] = jnp.dot(ea, w2_ref[...], preferred_element_type=jnp.float32)


def _edge_embed(ea_ext, We1, We2):
    blk = 512
    return pl.pallas_call(
        _ee_body,
        grid=(_E1PAD // blk,),
        in_specs=[
            pl.BlockSpec((blk, 16), lambda i: (i, 0)),
            pl.BlockSpec((16, 128), lambda i: (0, 0)),
            pl.BlockSpec((16, 128), lambda i: (0, 0)),
        ],
        out_specs=[
            pl.BlockSpec((blk, 128), lambda i: (i, 0)),
            pl.BlockSpec((blk, 128), lambda i: (i, 0)),
        ],
        out_shape=[
            jax.ShapeDtypeStruct((_E1PAD, 128), jnp.float32),
            jax.ShapeDtypeStruct((_E1PAD, 128), jnp.float32),
        ],
    )(ea_ext, We1, We2)


def _proj_body(x_ref, wl_ref, bl_ref, wr_ref, br_ref, xl_ref, xr_ref):
    x = x_ref[...]
    xl_ref[...] = jnp.dot(x, wl_ref[...],
                          preferred_element_type=jnp.float32) + bl_ref[...]
    xr_ref[...] = jnp.dot(x, wr_ref[...],
                          preferred_element_type=jnp.float32) + br_ref[...]


def _proj(x, Wl, bl, Wr, br):
    blk = 512
    din = x.shape[1]
    return pl.pallas_call(
        _proj_body,
        grid=(_NPAD // blk,),
        in_specs=[
            pl.BlockSpec((blk, din), lambda i: (i, 0)),
            pl.BlockSpec((din, 128), lambda i: (0, 0)),
            pl.BlockSpec((1, 128), lambda i: (0, 0)),
            pl.BlockSpec((din, 128), lambda i: (0, 0)),
            pl.BlockSpec((1, 128), lambda i: (0, 0)),
        ],
        out_specs=[
            pl.BlockSpec((blk, 128), lambda i: (i, 0)),
            pl.BlockSpec((blk, 128), lambda i: (i, 0)),
        ],
        out_shape=[
            jax.ShapeDtypeStruct((_NPAD, 128), jnp.float32),
            jax.ShapeDtypeStruct((_NPAD, 128), jnp.float32),
        ],
    )(x, Wl, bl.reshape(1, 128), Wr, br.reshape(1, 128))


def _pool_body(h_ref, batch_ref, wlin_ref, blin_ref, out_ref, acc_ref,
               cnt_ref):
    i = pl.program_id(0)

    @pl.when(i == 0)
    def _():
        acc_ref[...] = jnp.zeros_like(acc_ref)
        cnt_ref[...] = jnp.zeros_like(cnt_ref)

    blk = h_ref.shape[0]
    bb = batch_ref[...].reshape(1, blk)
    onehot = (lax.broadcasted_iota(jnp.int32, (_G, blk), 0) ==
              jnp.broadcast_to(bb, (_G, blk))).astype(jnp.float32)
    acc_ref[...] += jnp.dot(onehot, h_ref[...],
                            preferred_element_type=jnp.float32)
    cnt_ref[...] += jnp.sum(onehot, axis=1, keepdims=True)

    @pl.when(i == pl.num_programs(0) - 1)
    def _():
        pooled = acc_ref[...] / jnp.maximum(cnt_ref[...], 1.0)
        out = jnp.dot(pooled, wlin_ref[...],
                      preferred_element_type=jnp.float32) + blin_ref[...]
        out_ref[...] = jax.nn.sigmoid(out)


def _pool_final(h, batchp, Wlin, blin):
    blk = 512
    grid = _NPAD // blk
    return pl.pallas_call(
        _pool_body,
        grid=(grid,),
        in_specs=[
            pl.BlockSpec((blk, 128), lambda i: (i, 0)),
            pl.BlockSpec((1, 1, blk), lambda i: (i, 0, 0)),
            pl.BlockSpec((128, 1), lambda i: (0, 0)),
            pl.BlockSpec((1, 1), lambda i: (0, 0)),
        ],
        out_specs=pl.BlockSpec((_G, 1), lambda i: (0, 0)),
        out_shape=jax.ShapeDtypeStruct((_G, 1), jnp.float32),
        scratch_shapes=[
            pltpu.VMEM((_G, 128), jnp.float32),
            pltpu.VMEM((_G, 1), jnp.float32),
        ],
    )(h, batchp.reshape(grid, 1, blk), Wlin, blin.reshape(1, 1))


def _wid():
    return lax.axis_index("s") * _NC + lax.axis_index("c")


_GDN = lax.GatherDimensionNumbers(
    offset_dims=(), collapsed_slice_dims=(0,), start_index_map=(0,))


def _permute(v, idx):
    return lax.gather(v, idx[:, None], _GDN, (1,),
                      mode=lax.GatherScatterMode.PROMISE_IN_BOUNDS)


def _vsum(v, rots):
    for idx in rots:
        v = v + _permute(v, idx)
    return v[0]


def _make_rots():
    return [(jnp.arange(16, dtype=jnp.int32) + s) % 16 for s in (1, 2, 4, 8)]


_CHA = 128


def _alpha_kernel(H):
    mesh = plsc.VectorSubcoreMesh(core_axis_name="c", subcore_axis_name="s")

    def body(xl, xr, ee, srcs, dsts, permc, attf, alpha_out,
             srcv0, dstv0, permv0, gxl0, gxr0, gee0,
             srcv1, dstv1, permv1, gxl1, gxr1, gee1,
             attv, a0v, a1v, s1, s2, s3, s4, s5, s6):
        w = _wid()
        lane0 = lax.iota(jnp.int32, 16) == 0
        rots = _make_rots()
        pltpu.sync_copy(attf, attv)
        att_blk = [attv[pl.ds(cc * 16, 16)] for cc in range(8)]
        bufs = ((srcv0, dstv0, permv0, gxl0, gxr0, gee0, s1, s2, s3),
                (srcv1, dstv1, permv1, gxl1, gxr1, gee1, s4, s5, s6))

        def fire(kk, b):
            srcv, dstv, permv, gxl, gxr, gee, t1, t2, t3 = bufs[b]
            base = w * _EPT + kk * _CHA
            pltpu.sync_copy(srcs.at[pl.ds(base, _CHA)], srcv)
            pltpu.sync_copy(dsts.at[pl.ds(base, _CHA)], dstv)
            pltpu.sync_copy(permc.at[pl.ds(base, _CHA)], permv)
            return (pltpu.async_copy(xl.at[srcv], gxl, t1),
                    pltpu.async_copy(xr.at[dstv], gxr, t2),
                    pltpu.async_copy(ee.at[permv], gee, t3))

        def compute(kk, b):
            gxl, gxr, gee = bufs[b][3:6]
            base = w * _EPT + kk * _CHA

            def edge(e, carry2):
                vs0 = jnp.zeros((16,), jnp.float32)
                vs1 = jnp.zeros((16,), jnp.float32)
                for cc in range(8):
                    sl = pl.ds(cc * 16, 16)
                    v = gxl[e, sl] + gxr[e, sl] + gee[e, sl]
                    v = jnp.where(v > 0, v, 0.2 * v)
                    av = v * att_blk[cc]
                    if H == 2 and cc >= 4:
                        vs1 = vs1 + av
                    else:
                        vs0 = vs0 + av
                es = jnp.full((16,), e, jnp.int32)
                plsc.store_scatter(a0v, [es],
                                   jnp.full((16,), _vsum(vs0, rots)),
                                   mask=lane0)
                if H == 2:
                    plsc.store_scatter(a1v, [es],
                                       jnp.full((16,), _vsum(vs1, rots)),
                                       mask=lane0)
                return carry2

            lax.fori_loop(0, _CHA, edge, 0, unroll=2)
            pltpu.sync_copy(a0v, alpha_out.at[0, pl.ds(base, _CHA)])
            if H == 2:
                pltpu.sync_copy(a1v, alpha_out.at[1, pl.ds(base, _CHA)])

        def pair(p, carry):
            h0 = fire(2 * p, 0)
            h1 = fire(2 * p + 1, 1)
            for h in h0:
                h.wait()
            compute(2 * p, 0)
            for h in h1:
                h.wait()
            compute(2 * p + 1, 1)
            return carry

        lax.fori_loop(0, _EPT // _CHA // 2, pair, 0)

    return pl.kernel(
        body,
        out_type=jax.ShapeDtypeStruct((H, _EPAD), jnp.float32),
        mesh=mesh,
        compiler_params=pltpu.CompilerParams(needs_layout_passes=False),
        scratch_types=(
            [pltpu.VMEM((_CHA,), jnp.int32)] * 3 +
            [pltpu.VMEM((_CHA, 128), jnp.float32)] * 3 +
            [pltpu.VMEM((_CHA,), jnp.int32)] * 3 +
            [pltpu.VMEM((_CHA, 128), jnp.float32)] * 3 +
            [pltpu.VMEM((128,), jnp.float32),
             pltpu.VMEM((_CHA,), jnp.float32),
             pltpu.VMEM((_CHA,), jnp.float32)] +
            [pltpu.SemaphoreType.DMA] * 6
        ),
    )


def _agg_kernel(H, relu):
    mesh = plsc.VectorSubcoreMesh(core_axis_name="c", subcore_axis_name="s")

    def body(xl, alpha, srcs, dsts, rowptr, biasf, out,
             rpv, m0, m1, d0, d1, srcv, dstv, a0v, a1v, w0v, w1v, gxl,
             accv, stg, biasv, s1):
        w = _wid()
        n0 = w * _NPT
        lane0 = lax.iota(jnp.int32, 16) == 0
        pltpu.sync_copy(rowptr.at[pl.ds(n0, _RPS)], rpv)
        pltpu.sync_copy(biasf, biasv)

        def init(i, carry):
            m0[pl.ds(i * 16, 16)] = jnp.full((16,), -1e30, jnp.float32)
            m1[pl.ds(i * 16, 16)] = jnp.full((16,), -1e30, jnp.float32)
            d0[pl.ds(i * 16, 16)] = jnp.zeros((16,), jnp.float32)
            d1[pl.ds(i * 16, 16)] = jnp.zeros((16,), jnp.float32)
            return carry

        lax.fori_loop(0, (_NPT + 16) // 16, init, 0)

        def zrow(e, carry):
            for cc in range(8):
                gxl[e, pl.ds(cc * 16, 16)] = jnp.zeros((16,), jnp.float32)
            return carry

        lax.fori_loop(0, _CH, zrow, 0)

        def zfill(i, carry):
            pltpu.sync_copy(gxl, out.at[pl.ds(n0 + i * _CH, _CH), :])
            return carry

        lax.fori_loop(0, _NPT // _CH, zfill, 0)
        pltpu.sync_copy(gxl.at[pl.ds(0, _NPT % _CH), :],
                        out.at[pl.ds(n0 + (_NPT // _CH) * _CH,
                                     _NPT % _CH), :])

        rp0 = rpv[pl.ds(0, 16)][0]
        rp1 = rpv[pl.ds(_NPT, 16)][0]
        kstart = rp0 // _CH
        kend = (rp1 + _CH - 1) // _CH

        def stage(k, also_src):
            cb = k * _CH
            pltpu.sync_copy(dsts.at[pl.ds(cb, _CH)], dstv.at[pl.ds(0, _CH)])
            pltpu.sync_copy(alpha.at[0, pl.ds(cb, _CH)], a0v.at[pl.ds(0, _CH)])
            if H == 2:
                pltpu.sync_copy(alpha.at[1, pl.ds(cb, _CH)],
                                a1v.at[pl.ds(0, _CH)])
            if also_src:
                pltpu.sync_copy(srcs.at[pl.ds(cb, _CH)], srcv)
            lo = jnp.maximum(cb, rp0) - cb
            hi = jnp.minimum(cb + _CH, rp1) - cb
            return lo, hi

        def sweep_a(k, carry):
            lo, hi = stage(k, False)

            def per_edge(i, c2):
                nl = dstv[pl.ds(i, 16)][0] - n0
                nls = jnp.full((16,), nl, jnp.int32)
                av = a0v[pl.ds(i, 16)]
                mo = m0[pl.ds(nl, 16)]
                plsc.store_scatter(m0, [nls], jnp.maximum(mo, av), mask=lane0)
                if H == 2:
                    av1 = a1v[pl.ds(i, 16)]
                    mo1 = m1[pl.ds(nl, 16)]
                    plsc.store_scatter(m1, [nls], jnp.maximum(mo1, av1),
                                       mask=lane0)
                return c2

            lax.fori_loop(lo, hi, per_edge, 0)
            return carry

        lax.fori_loop(kstart, kend, sweep_a, 0)

        def sweep_b(k, carry):
            lo, hi = stage(k, False)

            def vec(g, c2):
                sl = pl.ds(g * 16, 16)
                nl = jnp.clip(dstv[sl] - n0, 0, _NPT - 1)
                w0v[sl] = jnp.exp(a0v[sl] - plsc.load_gather(m0, [nl]))
                if H == 2:
                    w1v[sl] = jnp.exp(a1v[sl] - plsc.load_gather(m1, [nl]))
                return c2

            lax.fori_loop(0, _CH // 16, vec, 0)

            def per_edge(i, c2):
                nl = dstv[pl.ds(i, 16)][0] - n0
                nls = jnp.full((16,), nl, jnp.int32)
                dv = d0[pl.ds(nl, 16)]
                plsc.store_scatter(d0, [nls], dv + w0v[pl.ds(i, 16)],
                                   mask=lane0)
                if H == 2:
                    dv1 = d1[pl.ds(nl, 16)]
                    plsc.store_scatter(d1, [nls], dv1 + w1v[pl.ds(i, 16)],
                                       mask=lane0)
                return c2

            lax.fori_loop(lo, hi, per_edge, 0)
            return carry

        lax.fori_loop(kstart, kend, sweep_b, 0)

        zero8 = tuple(jnp.zeros((16,), jnp.float32) for _ in range(8))

        def flush(cur, acc):
            for cc in range(8):
                sl = pl.ds(cc * 16, 16)
                v = acc[cc] + biasv[sl]
                if relu:
                    v = jnp.maximum(v, 0.0)
                stg[sl] = v
            pltpu.sync_copy(stg, out.at[cur])

        def sweep_c(k, carry):
            lo, hi = stage(k, True)
            cp = pltpu.async_copy(xl.at[srcv], gxl, s1)

            def vec(g, c2):
                sl = pl.ds(g * 16, 16)
                nl = jnp.clip(dstv[sl] - n0, 0, _NPT - 1)
                ex0 = jnp.exp(a0v[sl] - plsc.load_gather(m0, [nl]))
                w0v[sl] = ex0 / (plsc.load_gather(d0, [nl]) + 1e-16)
                if H == 2:
                    ex1 = jnp.exp(a1v[sl] - plsc.load_gather(m1, [nl]))
                    w1v[sl] = ex1 / (plsc.load_gather(d1, [nl]) + 1e-16)
                return c2

            lax.fori_loop(0, _CH // 16, vec, 0)
            cp.wait()

            def per_edge(i, carry2):
                cur2 = carry2[0]
                acc = carry2[1:]
                nd = dstv[pl.ds(i, 16)][0]
                changed = nd != cur2

                @pl.when(changed & (cur2 >= 0))
                def _():
                    flush(cur2, acc)

                w0s = w0v[pl.ds(i, 16)][0]
                if H == 2:
                    w1s = w1v[pl.ds(i, 16)][0]
                nacc = []
                for cc in range(8):
                    ws = w1s if (H == 2 and cc >= 4) else w0s
                    a = jnp.where(changed, 0.0, acc[cc])
                    nacc.append(a + ws * gxl[i, pl.ds(cc * 16, 16)])
                return (nd,) + tuple(nacc)

            return lax.fori_loop(lo, hi, per_edge, carry)

        carry = lax.fori_loop(kstart, kend, sweep_c,
                              (jnp.int32(-1),) + zero8)

        @pl.when(carry[0] >= 0)
        def _():
            flush(carry[0], carry[1:])

    return pl.kernel(
        body,
        out_type=jax.ShapeDtypeStruct((_NPAD, 128), jnp.float32),
        mesh=mesh,
        compiler_params=pltpu.CompilerParams(needs_layout_passes=False),
        scratch_types=[
            pltpu.VMEM((_RPS,), jnp.int32),
            pltpu.VMEM((_NPT + 16,), jnp.float32),
            pltpu.VMEM((_NPT + 16,), jnp.float32),
            pltpu.VMEM((_NPT + 16,), jnp.float32),
            pltpu.VMEM((_NPT + 16,), jnp.float32),
            pltpu.VMEM((_CH,), jnp.int32),
            pltpu.VMEM((_CH + 16,), jnp.int32),
            pltpu.VMEM((_CH + 16,), jnp.float32),
            pltpu.VMEM((_CH + 16,), jnp.float32),
            pltpu.VMEM((_CH + 16,), jnp.float32),
            pltpu.VMEM((_CH + 16,), jnp.float32),
            pltpu.VMEM((_CH, 128), jnp.float32),
            pltpu.VMEM((128,), jnp.float32),
            pltpu.VMEM((128,), jnp.float32),
            pltpu.VMEM((128,), jnp.float32),
            pltpu.SemaphoreType.DMA,
        ],
    )


def kernel(x, edge_index, edge_attr, batch, Wl1, bl1, Wr1, br1, We1, att1,
           bias1, Wl2, bl2, Wr2, br2, We2, att2, bias2, Wlin, blin):
    loop = jnp.arange(_N, dtype=jnp.int32)
    src2 = jnp.concatenate([edge_index[0], loop])
    dst2 = jnp.concatenate([edge_index[1], loop])
    perm = jnp.argsort(dst2).astype(jnp.int32)
    dst_s = dst2[perm]
    src_s = src2[perm]
    permc = jnp.minimum(perm, _E)

    srcs = jnp.zeros((_EPAD,), jnp.int32).at[:_E2].set(src_s)
    dsts = jnp.full((_EPAD,), _NPAD, jnp.int32).at[:_E2].set(dst_s)
    dstg = jnp.minimum(dsts, _NPAD - 1)
    permcp = jnp.zeros((_EPAD,), jnp.int32).at[:_E2].set(permc)
    rowptr = jnp.searchsorted(dsts, jnp.arange(_NPAD + 1, dtype=jnp.int32)
                              ).astype(jnp.int32)
    rowptr = jnp.concatenate(
        [rowptr, jnp.full((_NW * _NPT + _RPS - _NPAD - 1,), _E2, jnp.int32)])

    mean_row = _mean_rows(edge_attr)
    ea_ext = jnp.concatenate(
        [edge_attr, jnp.broadcast_to(mean_row, (_E1PAD - _E, 16))])
    ee1, ee2 = _edge_embed(ea_ext, We1, We2)

    xpad = jnp.zeros((_NPAD, x.shape[1]), jnp.float32).at[:_N].set(x)
    xl1, xr1 = _proj(xpad, Wl1, bl1, Wr1, br1)

    alpha1 = _alpha_kernel(2)(xl1, xr1, ee1, srcs, dstg, permcp,
                              att1.reshape(128))
    h = _agg_kernel(2, True)(xl1, alpha1, srcs, dsts, rowptr,
                             bias1.reshape(128))

    xl2, xr2 = _proj(h, Wl2, bl2, Wr2, br2)
    alpha2 = _alpha_kernel(1)(xl2, xr2, ee2, srcs, dstg, permcp,
                              att2.reshape(128))
    h2 = _agg_kernel(1, False)(xl2, alpha2, srcs, dsts, rowptr,
                               bias2.reshape(128))

    batchp = jnp.full((_NPAD,), _G, jnp.int32).at[:_N].set(batch)
    return _pool_final(h2, batchp, Wlin, blin)


# speedup vs baseline: 8.0493x; 1.0399x over previous
import functools

import jax
import jax.numpy as jnp
from jax import lax
from jax.experimental import pallas as pl
from jax.experimental.pallas import tpu as pltpu
from jax.experimental.pallas import tpu_sc as plsc

_N = 50000
_E = 800000
_E2 = _E + _N
_G = 64

_NC, _NS = 2, 16
_NW = _NC * _NS
_NPT = 1568
_NPAD = _NW * _NPT
_CH = 256
_EPT = 26624
_EPAD = _NW * _EPT
_E1PAD = 800256
_RPS = 1584


def _mean_body(ea_ref, out_ref, acc_ref):
    i = pl.program_id(0)

    @pl.when(i == 0)
    def _():
        acc_ref[...] = jnp.zeros_like(acc_ref)

    acc_ref[...] += ea_ref[...]

    @pl.when(i == pl.num_programs(0) - 1)
    def _():
        out_ref[...] = jnp.sum(acc_ref[...], axis=0, keepdims=True) / _E


def _mean_rows(ea):
    blk = 256
    return pl.pallas_call(
        _mean_body,
        grid=(_E // blk,),
        in_specs=[pl.BlockSpec((blk, 16), lambda i: (i, 0))],
        out_specs=pl.BlockSpec((1, 16), lambda i: (0, 0)),
        out_shape=jax.ShapeDtypeStruct((1, 16), jnp.float32),
        scratch_shapes=[pltpu.VMEM((blk, 16), jnp.float32)],
    )(ea)


def _ee_body(ea_ref, w1_ref, w2_ref, o1_ref, o2_ref):
    ea = ea_ref[...]
    o1_ref[...] = jnp.dot(ea, w1_ref[...], preferred_element_type=jnp.float32)
    o2_ref[...] = jnp.dot(ea, w2_ref[...], preferred_element_type=jnp.float32)


def _edge_embed(ea_ext, We1, We2):
    blk = 512
    return pl.pallas_call(
        _ee_body,
        grid=(_E1PAD // blk,),
        in_specs=[
            pl.BlockSpec((blk, 16), lambda i: (i, 0)),
            pl.BlockSpec((16, 128), lambda i: (0, 0)),
            pl.BlockSpec((16, 128), lambda i: (0, 0)),
        ],
        out_specs=[
            pl.BlockSpec((blk, 128), lambda i: (i, 0)),
            pl.BlockSpec((blk, 128), lambda i: (i, 0)),
        ],
        out_shape=[
            jax.ShapeDtypeStruct((_E1PAD, 128), jnp.float32),
            jax.ShapeDtypeStruct((_E1PAD, 128), jnp.float32),
        ],
    )(ea_ext, We1, We2)


def _proj_body(x_ref, wl_ref, bl_ref, wr_ref, br_ref, xl_ref, xr_ref):
    x = x_ref[...]
    xl_ref[...] = jnp.dot(x, wl_ref[...],
                          preferred_element_type=jnp.float32) + bl_ref[...]
    xr_ref[...] = jnp.dot(x, wr_ref[...],
                          preferred_element_type=jnp.float32) + br_ref[...]


def _proj(x, Wl, bl, Wr, br):
    blk = 512
    din = x.shape[1]
    return pl.pallas_call(
        _proj_body,
        grid=(_NPAD // blk,),
        in_specs=[
            pl.BlockSpec((blk, din), lambda i: (i, 0)),
            pl.BlockSpec((din, 128), lambda i: (0, 0)),
            pl.BlockSpec((1, 128), lambda i: (0, 0)),
            pl.BlockSpec((din, 128), lambda i: (0, 0)),
            pl.BlockSpec((1, 128), lambda i: (0, 0)),
        ],
        out_specs=[
            pl.BlockSpec((blk, 128), lambda i: (i, 0)),
            pl.BlockSpec((blk, 128), lambda i: (i, 0)),
        ],
        out_shape=[
            jax.ShapeDtypeStruct((_NPAD, 128), jnp.float32),
            jax.ShapeDtypeStruct((_NPAD, 128), jnp.float32),
        ],
    )(x, Wl, bl.reshape(1, 128), Wr, br.reshape(1, 128))


def _pool_body(h_ref, batch_ref, wlin_ref, blin_ref, out_ref, acc_ref,
               cnt_ref):
    i = pl.program_id(0)

    @pl.when(i == 0)
    def _():
        acc_ref[...] = jnp.zeros_like(acc_ref)
        cnt_ref[...] = jnp.zeros_like(cnt_ref)

    blk = h_ref.shape[0]
    bb = batch_ref[...].reshape(1, blk)
    onehot = (lax.broadcasted_iota(jnp.int32, (_G, blk), 0) ==
              jnp.broadcast_to(bb, (_G, blk))).astype(jnp.float32)
    acc_ref[...] += jnp.dot(onehot, h_ref[...],
                            preferred_element_type=jnp.float32)
    cnt_ref[...] += jnp.sum(onehot, axis=1, keepdims=True)

    @pl.when(i == pl.num_programs(0) - 1)
    def _():
        pooled = acc_ref[...] / jnp.maximum(cnt_ref[...], 1.0)
        out = jnp.dot(pooled, wlin_ref[...],
                      preferred_element_type=jnp.float32) + blin_ref[...]
        out_ref[...] = jax.nn.sigmoid(out)


def _pool_final(h, batchp, Wlin, blin):
    blk = 512
    grid = _NPAD // blk
    return pl.pallas_call(
        _pool_body,
        grid=(grid,),
        in_specs=[
            pl.BlockSpec((blk, 128), lambda i: (i, 0)),
            pl.BlockSpec((1, 1, blk), lambda i: (i, 0, 0)),
            pl.BlockSpec((128, 1), lambda i: (0, 0)),
            pl.BlockSpec((1, 1), lambda i: (0, 0)),
        ],
        out_specs=pl.BlockSpec((_G, 1), lambda i: (0, 0)),
        out_shape=jax.ShapeDtypeStruct((_G, 1), jnp.float32),
        scratch_shapes=[
            pltpu.VMEM((_G, 128), jnp.float32),
            pltpu.VMEM((_G, 1), jnp.float32),
        ],
    )(h, batchp.reshape(grid, 1, blk), Wlin, blin.reshape(1, 1))


def _wid():
    return lax.axis_index("s") * _NC + lax.axis_index("c")


_GDN = lax.GatherDimensionNumbers(
    offset_dims=(), collapsed_slice_dims=(0,), start_index_map=(0,))


def _permute(v, idx):
    return lax.gather(v, idx[:, None], _GDN, (1,),
                      mode=lax.GatherScatterMode.PROMISE_IN_BOUNDS)


def _vsum(v, rots):
    for idx in rots:
        v = v + _permute(v, idx)
    return v[0]


def _make_rots():
    return [(jnp.arange(16, dtype=jnp.int32) + s) % 16 for s in (1, 2, 4, 8)]


_CHA = 128
_SCH = 2048


def _alpha_kernel(H):
    mesh = plsc.VectorSubcoreMesh(core_axis_name="c", subcore_axis_name="s")

    def body(xl, xr, ee, srcs, dsts, permc, attf, alpha_out,
             srcb, dstb, permb, gxl0, gxr0, gee0, gxl1, gxr1, gee1,
             attv, a0b, a1b, s1, s2, s3, s4, s5, s6):
        w = _wid()
        lane0 = lax.iota(jnp.int32, 16) == 0
        rots = _make_rots()
        pltpu.sync_copy(attf, attv)
        att_blk = [attv[pl.ds(cc * 16, 16)] for cc in range(8)]
        bufs = ((gxl0, gxr0, gee0, s1, s2, s3),
                (gxl1, gxr1, gee1, s4, s5, s6))

        def super_(si, carry):
            sbase = w * _EPT + si * _SCH
            pltpu.sync_copy(srcs.at[pl.ds(sbase, _SCH)], srcb)
            pltpu.sync_copy(dsts.at[pl.ds(sbase, _SCH)], dstb)
            pltpu.sync_copy(permc.at[pl.ds(sbase, _SCH)], permb)

            def fire(j, b):
                gxl, gxr, gee, t1, t2, t3 = bufs[b]
                off = j * _CHA
                return (
                    pltpu.async_copy(xl.at[srcb.at[pl.ds(off, _CHA)]],
                                     gxl, t1),
                    pltpu.async_copy(xr.at[dstb.at[pl.ds(off, _CHA)]],
                                     gxr, t2),
                    pltpu.async_copy(ee.at[permb.at[pl.ds(off, _CHA)]],
                                     gee, t3))

            def compute(j, b):
                gxl, gxr, gee = bufs[b][:3]
                off = j * _CHA

                def edge(e, carry2):
                    vs0 = jnp.zeros((16,), jnp.float32)
                    vs1 = jnp.zeros((16,), jnp.float32)
                    for cc in range(8):
                        sl = pl.ds(cc * 16, 16)
                        v = gxl[e, sl] + gxr[e, sl] + gee[e, sl]
                        v = jnp.where(v > 0, v, 0.2 * v)
                        av = v * att_blk[cc]
                        if H == 2 and cc >= 4:
                            vs1 = vs1 + av
                        else:
                            vs0 = vs0 + av
                    es = jnp.full((16,), off + e, jnp.int32)
                    plsc.store_scatter(a0b, [es],
                                       jnp.full((16,), _vsum(vs0, rots)),
                                       mask=lane0)
                    if H == 2:
                        plsc.store_scatter(a1b, [es],
                                           jnp.full((16,), _vsum(vs1, rots)),
                                           mask=lane0)
                    return carry2

                lax.fori_loop(0, _CHA, edge, 0, unroll=2)

            def pair(p, carry2):
                h0 = fire(2 * p, 0)
                h1 = fire(2 * p + 1, 1)
                for h in h0:
                    h.wait()
                compute(2 * p, 0)
                for h in h1:
                    h.wait()
                compute(2 * p + 1, 1)
                return carry2

            lax.fori_loop(0, _SCH // _CHA // 2, pair, 0)
            pltpu.sync_copy(a0b, alpha_out.at[0, pl.ds(sbase, _SCH)])
            if H == 2:
                pltpu.sync_copy(a1b, alpha_out.at[1, pl.ds(sbase, _SCH)])
            return carry

        lax.fori_loop(0, _EPT // _SCH, super_, 0)

    return pl.kernel(
        body,
        out_type=jax.ShapeDtypeStruct((H, _EPAD), jnp.float32),
        mesh=mesh,
        compiler_params=pltpu.CompilerParams(needs_layout_passes=False),
        scratch_types=(
            [pltpu.VMEM((_SCH,), jnp.int32)] * 3 +
            [pltpu.VMEM((_CHA, 128), jnp.float32)] * 6 +
            [pltpu.VMEM((128,), jnp.float32),
             pltpu.VMEM((_SCH,), jnp.float32),
             pltpu.VMEM((_SCH,), jnp.float32)] +
            [pltpu.SemaphoreType.DMA] * 6
        ),
    )


def _agg_kernel(H, relu):
    mesh = plsc.VectorSubcoreMesh(core_axis_name="c", subcore_axis_name="s")

    def body(xl, alpha, srcs, dsts, rowptr, biasf, out,
             rpv, m0, m1, d0, d1, srcv, dstv, a0v, a1v, w0v, w1v,
             gxl, gxlb, stg, biasv, s1, s2):
        w = _wid()
        n0 = w * _NPT
        lane0 = lax.iota(jnp.int32, 16) == 0
        pltpu.sync_copy(rowptr.at[pl.ds(n0, _RPS)], rpv)
        pltpu.sync_copy(biasf, biasv)

        def init(i, carry):
            m0[pl.ds(i * 16, 16)] = jnp.full((16,), -1e30, jnp.float32)
            m1[pl.ds(i * 16, 16)] = jnp.full((16,), -1e30, jnp.float32)
            d0[pl.ds(i * 16, 16)] = jnp.zeros((16,), jnp.float32)
            d1[pl.ds(i * 16, 16)] = jnp.zeros((16,), jnp.float32)
            return carry

        lax.fori_loop(0, (_NPT + 16) // 16, init, 0)

        def zrow(e, carry):
            for cc in range(8):
                gxl[e, pl.ds(cc * 16, 16)] = jnp.zeros((16,), jnp.float32)
            return carry

        lax.fori_loop(0, _CH, zrow, 0)

        def zfill(i, carry):
            pltpu.sync_copy(gxl, out.at[pl.ds(n0 + i * _CH, _CH), :])
            return carry

        lax.fori_loop(0, _NPT // _CH, zfill, 0)
        pltpu.sync_copy(gxl.at[pl.ds(0, _NPT % _CH), :],
                        out.at[pl.ds(n0 + (_NPT // _CH) * _CH,
                                     _NPT % _CH), :])

        rp0 = rpv[pl.ds(0, 16)][0]
        rp1 = rpv[pl.ds(_NPT, 16)][0]
        kstart = rp0 // _SCH
        kend = (rp1 + _SCH - 1) // _SCH

        def stage(k, also_src):
            cb = k * _SCH
            pltpu.sync_copy(dsts.at[pl.ds(cb, _SCH)], dstv.at[pl.ds(0, _SCH)])
            pltpu.sync_copy(alpha.at[0, pl.ds(cb, _SCH)],
                            a0v.at[pl.ds(0, _SCH)])
            if H == 2:
                pltpu.sync_copy(alpha.at[1, pl.ds(cb, _SCH)],
                                a1v.at[pl.ds(0, _SCH)])
            if also_src:
                pltpu.sync_copy(srcs.at[pl.ds(cb, _SCH)], srcv)
            lo = jnp.maximum(cb, rp0) - cb
            hi = jnp.minimum(cb + _SCH, rp1) - cb
            return lo, hi

        def sweep_a(k, carry):
            lo, hi = stage(k, False)

            def per_edge(i, c2):
                nl = dstv[pl.ds(i, 16)][0] - n0
                nls = jnp.full((16,), nl, jnp.int32)
                av = a0v[pl.ds(i, 16)]
                mo = m0[pl.ds(nl, 16)]
                plsc.store_scatter(m0, [nls], jnp.maximum(mo, av), mask=lane0)
                if H == 2:
                    av1 = a1v[pl.ds(i, 16)]
                    mo1 = m1[pl.ds(nl, 16)]
                    plsc.store_scatter(m1, [nls], jnp.maximum(mo1, av1),
                                       mask=lane0)
                return c2

            lax.fori_loop(lo, hi, per_edge, 0)
            return carry

        lax.fori_loop(kstart, kend, sweep_a, 0)

        def sweep_b(k, carry):
            lo, hi = stage(k, False)

            def vec(g, c2):
                sl = pl.ds(g * 16, 16)
                nl = jnp.clip(dstv[sl] - n0, 0, _NPT - 1)
                w0v[sl] = jnp.exp(a0v[sl] - plsc.load_gather(m0, [nl]))
                if H == 2:
                    w1v[sl] = jnp.exp(a1v[sl] - plsc.load_gather(m1, [nl]))
                return c2

            lax.fori_loop(0, _SCH // 16, vec, 0)

            def per_edge(i, c2):
                nl = dstv[pl.ds(i, 16)][0] - n0
                nls = jnp.full((16,), nl, jnp.int32)
                dv = d0[pl.ds(nl, 16)]
                plsc.store_scatter(d0, [nls], dv + w0v[pl.ds(i, 16)],
                                   mask=lane0)
                if H == 2:
                    dv1 = d1[pl.ds(nl, 16)]
                    plsc.store_scatter(d1, [nls], dv1 + w1v[pl.ds(i, 16)],
                                       mask=lane0)
                return c2

            lax.fori_loop(lo, hi, per_edge, 0)
            return carry

        lax.fori_loop(kstart, kend, sweep_b, 0)

        zero8 = tuple(jnp.zeros((16,), jnp.float32) for _ in range(8))

        def flush(cur, acc):
            for cc in range(8):
                sl = pl.ds(cc * 16, 16)
                v = acc[cc] + biasv[sl]
                if relu:
                    v = jnp.maximum(v, 0.0)
                stg[sl] = v
            pltpu.sync_copy(stg, out.at[cur])

        def sweep_c(k, carry):
            lo, hi = stage(k, True)

            def vec(g, c2):
                sl = pl.ds(g * 16, 16)
                nl = jnp.clip(dstv[sl] - n0, 0, _NPT - 1)
                ex0 = jnp.exp(a0v[sl] - plsc.load_gather(m0, [nl]))
                w0v[sl] = ex0 / (plsc.load_gather(d0, [nl]) + 1e-16)
                if H == 2:
                    ex1 = jnp.exp(a1v[sl] - plsc.load_gather(m1, [nl]))
                    w1v[sl] = ex1 / (plsc.load_gather(d1, [nl]) + 1e-16)
                return c2

            lax.fori_loop(0, _SCH // 16, vec, 0)

            def run_block(sb, gbuf, carry2):
                base = sb * _CH
                lo2 = jnp.maximum(lo, base)
                hi2 = jnp.minimum(hi, base + _CH)

                def per_edge(i, carry3):
                    cur3 = carry3[0]
                    acc = carry3[1:]
                    nd = dstv[pl.ds(i, 16)][0]
                    changed = nd != cur3

                    @pl.when(changed & (cur3 >= 0))
                    def _():
                        flush(cur3, acc)

                    w0s = w0v[pl.ds(i, 16)][0]
                    if H == 2:
                        w1s = w1v[pl.ds(i, 16)][0]
                    nacc = []
                    for cc in range(8):
                        ws = w1s if (H == 2 and cc >= 4) else w0s
                        a = jnp.where(changed, 0.0, acc[cc])
                        nacc.append(a + ws * gbuf[i - base,
                                                  pl.ds(cc * 16, 16)])
                    return (nd,) + tuple(nacc)

                return lax.fori_loop(lo2, hi2, per_edge, carry2)

            def sub(q, carry2):
                hA = pltpu.async_copy(
                    xl.at[srcv.at[pl.ds((2 * q) * _CH, _CH)]], gxl, s1)
                hB = pltpu.async_copy(
                    xl.at[srcv.at[pl.ds((2 * q + 1) * _CH, _CH)]], gxlb, s2)
                hA.wait()
                carry2 = run_block(2 * q, gxl, carry2)
                hB.wait()
                return run_block(2 * q + 1, gxlb, carry2)

            return lax.fori_loop(0, _SCH // _CH // 2, sub, carry)

        carry = lax.fori_loop(kstart, kend, sweep_c,
                              (jnp.int32(-1),) + zero8)

        @pl.when(carry[0] >= 0)
        def _():
            flush(carry[0], carry[1:])

    return pl.kernel(
        body,
        out_type=jax.ShapeDtypeStruct((_NPAD, 128), jnp.float32),
        mesh=mesh,
        compiler_params=pltpu.CompilerParams(needs_layout_passes=False),
        scratch_types=[
            pltpu.VMEM((_RPS,), jnp.int32),
            pltpu.VMEM((_NPT + 16,), jnp.float32),
            pltpu.VMEM((_NPT + 16,), jnp.float32),
            pltpu.VMEM((_NPT + 16,), jnp.float32),
            pltpu.VMEM((_NPT + 16,), jnp.float32),
            pltpu.VMEM((_SCH,), jnp.int32),
            pltpu.VMEM((_SCH + 16,), jnp.int32),
            pltpu.VMEM((_SCH + 16,), jnp.float32),
            pltpu.VMEM((_SCH + 16,), jnp.float32),
            pltpu.VMEM((_SCH + 16,), jnp.float32),
            pltpu.VMEM((_SCH + 16,), jnp.float32),
            pltpu.VMEM((_CH, 128), jnp.float32),
            pltpu.VMEM((_CH, 128), jnp.float32),
            pltpu.VMEM((128,), jnp.float32),
            pltpu.VMEM((128,), jnp.float32),
            pltpu.SemaphoreType.DMA,
            pltpu.SemaphoreType.DMA,
        ],
    )


def kernel(x, edge_index, edge_attr, batch, Wl1, bl1, Wr1, br1, We1, att1,
           bias1, Wl2, bl2, Wr2, br2, We2, att2, bias2, Wlin, blin):
    loop = jnp.arange(_N, dtype=jnp.int32)
    src2 = jnp.concatenate([edge_index[0], loop])
    dst2 = jnp.concatenate([edge_index[1], loop])
    perm = jnp.argsort(dst2).astype(jnp.int32)
    dst_s = dst2[perm]
    src_s = src2[perm]
    permc = jnp.minimum(perm, _E)

    srcs = jnp.zeros((_EPAD,), jnp.int32).at[:_E2].set(src_s)
    dsts = jnp.full((_EPAD,), _NPAD, jnp.int32).at[:_E2].set(dst_s)
    dstg = jnp.minimum(dsts, _NPAD - 1)
    permcp = jnp.zeros((_EPAD,), jnp.int32).at[:_E2].set(permc)
    rowptr = jnp.searchsorted(dsts, jnp.arange(_NPAD + 1, dtype=jnp.int32)
                              ).astype(jnp.int32)
    rowptr = jnp.concatenate(
        [rowptr, jnp.full((_NW * _NPT + _RPS - _NPAD - 1,), _E2, jnp.int32)])

    mean_row = _mean_rows(edge_attr)
    ea_ext = jnp.concatenate(
        [edge_attr, jnp.broadcast_to(mean_row, (_E1PAD - _E, 16))])
    ee1, ee2 = _edge_embed(ea_ext, We1, We2)

    xpad = jnp.zeros((_NPAD, x.shape[1]), jnp.float32).at[:_N].set(x)
    xl1, xr1 = _proj(xpad, Wl1, bl1, Wr1, br1)

    alpha1 = _alpha_kernel(2)(xl1, xr1, ee1, srcs, dstg, permcp,
                              att1.reshape(128))
    h = _agg_kernel(2, True)(xl1, alpha1, srcs, dsts, rowptr,
                             bias1.reshape(128))

    xl2, xr2 = _proj(h, Wl2, bl2, Wr2, br2)
    alpha2 = _alpha_kernel(1)(xl2, xr2, ee2, srcs, dstg, permcp,
                              att2.reshape(128))
    h2 = _agg_kernel(1, False)(xl2, alpha2, srcs, dsts, rowptr,
                               bias2.reshape(128))

    batchp = jnp.full((_NPAD,), _G, jnp.int32).at[:_N].set(batch)
    return _pool_final(h2, batchp, Wlin, blin)


# speedup vs baseline: 8.0519x; 1.0003x over previous
import functools

import jax
import jax.numpy as jnp
from jax import lax
from jax.experimental import pallas as pl
from jax.experimental.pallas import tpu as pltpu
from jax.experimental.pallas import tpu_sc as plsc

_N = 50000
_E = 800000
_E2 = _E + _N
_G = 64

_NC, _NS = 2, 16
_NW = _NC * _NS
_NPT = 1568
_NPAD = _NW * _NPT
_CH = 256
_EPT = 26624
_EPAD = _NW * _EPT
_E1PAD = 800256
_RPS = 1584


def _mean_body(ea_ref, out_ref, acc_ref):
    i = pl.program_id(0)

    @pl.when(i == 0)
    def _():
        acc_ref[...] = jnp.zeros_like(acc_ref)

    acc_ref[...] += ea_ref[...]

    @pl.when(i == pl.num_programs(0) - 1)
    def _():
        out_ref[...] = jnp.sum(acc_ref[...], axis=0, keepdims=True) / _E


def _mean_rows(ea):
    blk = 256
    return pl.pallas_call(
        _mean_body,
        grid=(_E // blk,),
        in_specs=[pl.BlockSpec((blk, 16), lambda i: (i, 0))],
        out_specs=pl.BlockSpec((1, 16), lambda i: (0, 0)),
        out_shape=jax.ShapeDtypeStruct((1, 16), jnp.float32),
        scratch_shapes=[pltpu.VMEM((blk, 16), jnp.float32)],
    )(ea)


def _ee_body(ea_ref, w1_ref, w2_ref, o1_ref, o2_ref):
    ea = ea_ref[...]
    o1_ref[...] = jnp.dot(ea, w1_ref[...], preferred_element_type=jnp.float32)
    o2_ref[...] = jnp.dot(ea, w2_ref[...], preferred_element_type=jnp.float32)


def _edge_embed(ea_ext, We1, We2):
    blk = 512
    return pl.pallas_call(
        _ee_body,
        grid=(_E1PAD // blk,),
        in_specs=[
            pl.BlockSpec((blk, 16), lambda i: (i, 0)),
            pl.BlockSpec((16, 128), lambda i: (0, 0)),
            pl.BlockSpec((16, 128), lambda i: (0, 0)),
        ],
        out_specs=[
            pl.BlockSpec((blk, 128), lambda i: (i, 0)),
            pl.BlockSpec((blk, 128), lambda i: (i, 0)),
        ],
        out_shape=[
            jax.ShapeDtypeStruct((_E1PAD, 128), jnp.float32),
            jax.ShapeDtypeStruct((_E1PAD, 128), jnp.float32),
        ],
    )(ea_ext, We1, We2)


def _proj_body(x_ref, wl_ref, bl_ref, wr_ref, br_ref, xl_ref, xr_ref):
    x = x_ref[...]
    xl_ref[...] = jnp.dot(x, wl_ref[...],
                          preferred_element_type=jnp.float32) + bl_ref[...]
    xr_ref[...] = jnp.dot(x, wr_ref[...],
                          preferred_element_type=jnp.float32) + br_ref[...]


def _proj(x, Wl, bl, Wr, br):
    blk = 512
    din = x.shape[1]
    return pl.pallas_call(
        _proj_body,
        grid=(_NPAD // blk,),
        in_specs=[
            pl.BlockSpec((blk, din), lambda i: (i, 0)),
            pl.BlockSpec((din, 128), lambda i: (0, 0)),
            pl.BlockSpec((1, 128), lambda i: (0, 0)),
            pl.BlockSpec((din, 128), lambda i: (0, 0)),
            pl.BlockSpec((1, 128), lambda i: (0, 0)),
        ],
        out_specs=[
            pl.BlockSpec((blk, 128), lambda i: (i, 0)),
            pl.BlockSpec((blk, 128), lambda i: (i, 0)),
        ],
        out_shape=[
            jax.ShapeDtypeStruct((_NPAD, 128), jnp.float32),
            jax.ShapeDtypeStruct((_NPAD, 128), jnp.float32),
        ],
    )(x, Wl, bl.reshape(1, 128), Wr, br.reshape(1, 128))


def _pool_body(h_ref, batch_ref, wlin_ref, blin_ref, out_ref, acc_ref,
               cnt_ref):
    i = pl.program_id(0)

    @pl.when(i == 0)
    def _():
        acc_ref[...] = jnp.zeros_like(acc_ref)
        cnt_ref[...] = jnp.zeros_like(cnt_ref)

    blk = h_ref.shape[0]
    bb = batch_ref[...].reshape(1, blk)
    onehot = (lax.broadcasted_iota(jnp.int32, (_G, blk), 0) ==
              jnp.broadcast_to(bb, (_G, blk))).astype(jnp.float32)
    acc_ref[...] += jnp.dot(onehot, h_ref[...],
                            preferred_element_type=jnp.float32)
    cnt_ref[...] += jnp.sum(onehot, axis=1, keepdims=True)

    @pl.when(i == pl.num_programs(0) - 1)
    def _():
        pooled = acc_ref[...] / jnp.maximum(cnt_ref[...], 1.0)
        out = jnp.dot(pooled, wlin_ref[...],
                      preferred_element_type=jnp.float32) + blin_ref[...]
        out_ref[...] = jax.nn.sigmoid(out)


def _pool_final(h, batchp, Wlin, blin):
    blk = 512
    grid = _NPAD // blk
    return pl.pallas_call(
        _pool_body,
        grid=(grid,),
        in_specs=[
            pl.BlockSpec((blk, 128), lambda i: (i, 0)),
            pl.BlockSpec((1, 1, blk), lambda i: (i, 0, 0)),
            pl.BlockSpec((128, 1), lambda i: (0, 0)),
            pl.BlockSpec((1, 1), lambda i: (0, 0)),
        ],
        out_specs=pl.BlockSpec((_G, 1), lambda i: (0, 0)),
        out_shape=jax.ShapeDtypeStruct((_G, 1), jnp.float32),
        scratch_shapes=[
            pltpu.VMEM((_G, 128), jnp.float32),
            pltpu.VMEM((_G, 1), jnp.float32),
        ],
    )(h, batchp.reshape(grid, 1, blk), Wlin, blin.reshape(1, 1))


def _wid():
    return lax.axis_index("s") * _NC + lax.axis_index("c")


_GDN = lax.GatherDimensionNumbers(
    offset_dims=(), collapsed_slice_dims=(0,), start_index_map=(0,))


def _permute(v, idx):
    return lax.gather(v, idx[:, None], _GDN, (1,),
                      mode=lax.GatherScatterMode.PROMISE_IN_BOUNDS)


def _vsum(v, rots):
    for idx in rots:
        v = v + _permute(v, idx)
    return v


def _make_rots():
    return [(jnp.arange(16, dtype=jnp.int32) + s) % 16 for s in (1, 2, 4, 8)]


_CHA = 128
_SCH = 2048


def _alpha_kernel(H):
    mesh = plsc.VectorSubcoreMesh(core_axis_name="c", subcore_axis_name="s")

    def body(xl, xr, ee, srcs, dsts, permc, attf, alpha_out,
             srcb, dstb, permb, gxl0, gxr0, gee0, gxl1, gxr1, gee1,
             attv, a0b, a1b, s1, s2, s3, s4, s5, s6):
        w = _wid()
        lane0 = lax.iota(jnp.int32, 16) == 0
        rots = _make_rots()
        pltpu.sync_copy(attf, attv)
        att_blk = [attv[pl.ds(cc * 16, 16)] for cc in range(8)]
        bufs = ((gxl0, gxr0, gee0, s1, s2, s3),
                (gxl1, gxr1, gee1, s4, s5, s6))

        def super_(si, carry):
            sbase = w * _EPT + si * _SCH
            pltpu.sync_copy(srcs.at[pl.ds(sbase, _SCH)], srcb)
            pltpu.sync_copy(dsts.at[pl.ds(sbase, _SCH)], dstb)
            pltpu.sync_copy(permc.at[pl.ds(sbase, _SCH)], permb)

            def fire(j, b):
                gxl, gxr, gee, t1, t2, t3 = bufs[b]
                off = j * _CHA
                return (
                    pltpu.async_copy(xl.at[srcb.at[pl.ds(off, _CHA)]],
                                     gxl, t1),
                    pltpu.async_copy(xr.at[dstb.at[pl.ds(off, _CHA)]],
                                     gxr, t2),
                    pltpu.async_copy(ee.at[permb.at[pl.ds(off, _CHA)]],
                                     gee, t3))

            def compute(j, b):
                gxl, gxr, gee = bufs[b][:3]
                off = j * _CHA
                lanes = lax.iota(jnp.int32, 16)

                def group(g, carry2):
                    av0 = jnp.zeros((16,), jnp.float32)
                    av1 = jnp.zeros((16,), jnp.float32)
                    for jj in range(16):
                        e = g * 16 + jj
                        vs0 = jnp.zeros((16,), jnp.float32)
                        vs1 = jnp.zeros((16,), jnp.float32)
                        for cc in range(8):
                            sl = pl.ds(cc * 16, 16)
                            v = gxl[e, sl] + gxr[e, sl] + gee[e, sl]
                            v = jnp.where(v > 0, v, 0.2 * v)
                            av = v * att_blk[cc]
                            if H == 2 and cc >= 4:
                                vs1 = vs1 + av
                            else:
                                vs0 = vs0 + av
                        ljj = lanes == jj
                        av0 = jnp.where(ljj, _vsum(vs0, rots), av0)
                        if H == 2:
                            av1 = jnp.where(ljj, _vsum(vs1, rots), av1)
                    a0b[pl.ds(off + g * 16, 16)] = av0
                    if H == 2:
                        a1b[pl.ds(off + g * 16, 16)] = av1
                    return carry2

                lax.fori_loop(0, _CHA // 16, group, 0)

            def pair(p, carry2):
                h0 = fire(2 * p, 0)
                h1 = fire(2 * p + 1, 1)
                for h in h0:
                    h.wait()
                compute(2 * p, 0)
                for h in h1:
                    h.wait()
                compute(2 * p + 1, 1)
                return carry2

            lax.fori_loop(0, _SCH // _CHA // 2, pair, 0)
            pltpu.sync_copy(a0b, alpha_out.at[0, pl.ds(sbase, _SCH)])
            if H == 2:
                pltpu.sync_copy(a1b, alpha_out.at[1, pl.ds(sbase, _SCH)])
            return carry

        lax.fori_loop(0, _EPT // _SCH, super_, 0)

    return pl.kernel(
        body,
        out_type=jax.ShapeDtypeStruct((H, _EPAD), jnp.float32),
        mesh=mesh,
        compiler_params=pltpu.CompilerParams(needs_layout_passes=False),
        scratch_types=(
            [pltpu.VMEM((_SCH,), jnp.int32)] * 3 +
            [pltpu.VMEM((_CHA, 128), jnp.float32)] * 6 +
            [pltpu.VMEM((128,), jnp.float32),
             pltpu.VMEM((_SCH,), jnp.float32),
             pltpu.VMEM((_SCH,), jnp.float32)] +
            [pltpu.SemaphoreType.DMA] * 6
        ),
    )


def _agg_kernel(H, relu):
    mesh = plsc.VectorSubcoreMesh(core_axis_name="c", subcore_axis_name="s")

    def body(xl, alpha, srcs, dsts, rowptr, biasf, out,
             rpv, m0, m1, d0, d1, srcv, dstv, a0v, a1v, w0v, w1v,
             gxl, gxlb, stg, biasv, s1, s2):
        w = _wid()
        n0 = w * _NPT
        lane0 = lax.iota(jnp.int32, 16) == 0
        pltpu.sync_copy(rowptr.at[pl.ds(n0, _RPS)], rpv)
        pltpu.sync_copy(biasf, biasv)

        def init(i, carry):
            m0[pl.ds(i * 16, 16)] = jnp.full((16,), -1e30, jnp.float32)
            m1[pl.ds(i * 16, 16)] = jnp.full((16,), -1e30, jnp.float32)
            d0[pl.ds(i * 16, 16)] = jnp.zeros((16,), jnp.float32)
            d1[pl.ds(i * 16, 16)] = jnp.zeros((16,), jnp.float32)
            return carry

        lax.fori_loop(0, (_NPT + 16) // 16, init, 0)

        def zrow(e, carry):
            for cc in range(8):
                gxl[e, pl.ds(cc * 16, 16)] = jnp.zeros((16,), jnp.float32)
            return carry

        lax.fori_loop(0, _CH, zrow, 0)

        def zfill(i, carry):
            pltpu.sync_copy(gxl, out.at[pl.ds(n0 + i * _CH, _CH), :])
            return carry

        lax.fori_loop(0, _NPT // _CH, zfill, 0)
        pltpu.sync_copy(gxl.at[pl.ds(0, _NPT % _CH), :],
                        out.at[pl.ds(n0 + (_NPT // _CH) * _CH,
                                     _NPT % _CH), :])

        rp0 = rpv[pl.ds(0, 16)][0]
        rp1 = rpv[pl.ds(_NPT, 16)][0]
        kstart = rp0 // _SCH
        kend = (rp1 + _SCH - 1) // _SCH

        def stage(k, also_src):
            cb = k * _SCH
            pltpu.sync_copy(dsts.at[pl.ds(cb, _SCH)], dstv.at[pl.ds(0, _SCH)])
            pltpu.sync_copy(alpha.at[0, pl.ds(cb, _SCH)],
                            a0v.at[pl.ds(0, _SCH)])
            if H == 2:
                pltpu.sync_copy(alpha.at[1, pl.ds(cb, _SCH)],
                                a1v.at[pl.ds(0, _SCH)])
            if also_src:
                pltpu.sync_copy(srcs.at[pl.ds(cb, _SCH)], srcv)
            lo = jnp.maximum(cb, rp0) - cb
            hi = jnp.minimum(cb + _SCH, rp1) - cb
            return lo, hi

        def sweep_a(k, carry):
            lo, hi = stage(k, False)

            def per_edge(i, c2):
                nl = dstv[pl.ds(i, 16)][0] - n0
                nls = jnp.full((16,), nl, jnp.int32)
                av = a0v[pl.ds(i, 16)]
                mo = m0[pl.ds(nl, 16)]
                plsc.store_scatter(m0, [nls], jnp.maximum(mo, av), mask=lane0)
                if H == 2:
                    av1 = a1v[pl.ds(i, 16)]
                    mo1 = m1[pl.ds(nl, 16)]
                    plsc.store_scatter(m1, [nls], jnp.maximum(mo1, av1),
                                       mask=lane0)
                return c2

            lax.fori_loop(lo, hi, per_edge, 0)
            return carry

        lax.fori_loop(kstart, kend, sweep_a, 0)

        def sweep_b(k, carry):
            lo, hi = stage(k, False)

            def vec(g, c2):
                sl = pl.ds(g * 16, 16)
                nl = jnp.clip(dstv[sl] - n0, 0, _NPT - 1)
                w0v[sl] = jnp.exp(a0v[sl] - plsc.load_gather(m0, [nl]))
                if H == 2:
                    w1v[sl] = jnp.exp(a1v[sl] - plsc.load_gather(m1, [nl]))
                return c2

            lax.fori_loop(0, _SCH // 16, vec, 0)

            def per_edge(i, c2):
                nl = dstv[pl.ds(i, 16)][0] - n0
                nls = jnp.full((16,), nl, jnp.int32)
                dv = d0[pl.ds(nl, 16)]
                plsc.store_scatter(d0, [nls], dv + w0v[pl.ds(i, 16)],
                                   mask=lane0)
                if H == 2:
                    dv1 = d1[pl.ds(nl, 16)]
                    plsc.store_scatter(d1, [nls], dv1 + w1v[pl.ds(i, 16)],
                                       mask=lane0)
                return c2

            lax.fori_loop(lo, hi, per_edge, 0)
            return carry

        lax.fori_loop(kstart, kend, sweep_b, 0)

        zero8 = tuple(jnp.zeros((16,), jnp.float32) for _ in range(8))

        def flush(cur, acc):
            for cc in range(8):
                sl = pl.ds(cc * 16, 16)
                v = acc[cc] + biasv[sl]
                if relu:
                    v = jnp.maximum(v, 0.0)
                stg[sl] = v
            pltpu.sync_copy(stg, out.at[cur])

        def sweep_c(k, carry):
            lo, hi = stage(k, True)

            def vec(g, c2):
                sl = pl.ds(g * 16, 16)
                nl = jnp.clip(dstv[sl] - n0, 0, _NPT - 1)
                ex0 = jnp.exp(a0v[sl] - plsc.load_gather(m0, [nl]))
                w0v[sl] = ex0 / (plsc.load_gather(d0, [nl]) + 1e-16)
                if H == 2:
                    ex1 = jnp.exp(a1v[sl] - plsc.load_gather(m1, [nl]))
                    w1v[sl] = ex1 / (plsc.load_gather(d1, [nl]) + 1e-16)
                return c2

            lax.fori_loop(0, _SCH // 16, vec, 0)

            def run_block(sb, gbuf, carry2):
                base = sb * _CH
                lo2 = jnp.maximum(lo, base)
                hi2 = jnp.minimum(hi, base + _CH)

                def per_edge(i, carry3):
                    cur3 = carry3[0]
                    acc = carry3[1:]
                    nd = dstv[pl.ds(i, 16)][0]
                    changed = nd != cur3

                    @pl.when(changed & (cur3 >= 0))
                    def _():
                        flush(cur3, acc)

                    w0s = w0v[pl.ds(i, 16)][0]
                    if H == 2:
                        w1s = w1v[pl.ds(i, 16)][0]
                    nacc = []
                    for cc in range(8):
                        ws = w1s if (H == 2 and cc >= 4) else w0s
                        a = jnp.where(changed, 0.0, acc[cc])
                        nacc.append(a + ws * gbuf[i - base,
                                                  pl.ds(cc * 16, 16)])
                    return (nd,) + tuple(nacc)

                return lax.fori_loop(lo2, hi2, per_edge, carry2)

            def sub(q, carry2):
                hA = pltpu.async_copy(
                    xl.at[srcv.at[pl.ds((2 * q) * _CH, _CH)]], gxl, s1)
                hB = pltpu.async_copy(
                    xl.at[srcv.at[pl.ds((2 * q + 1) * _CH, _CH)]], gxlb, s2)
                hA.wait()
                carry2 = run_block(2 * q, gxl, carry2)
                hB.wait()
                return run_block(2 * q + 1, gxlb, carry2)

            return lax.fori_loop(0, _SCH // _CH // 2, sub, carry)

        carry = lax.fori_loop(kstart, kend, sweep_c,
                              (jnp.int32(-1),) + zero8)

        @pl.when(carry[0] >= 0)
        def _():
            flush(carry[0], carry[1:])

    return pl.kernel(
        body,
        out_type=jax.ShapeDtypeStruct((_NPAD, 128), jnp.float32),
        mesh=mesh,
        compiler_params=pltpu.CompilerParams(needs_layout_passes=False),
        scratch_types=[
            pltpu.VMEM((_RPS,), jnp.int32),
            pltpu.VMEM((_NPT + 16,), jnp.float32),
            pltpu.VMEM((_NPT + 16,), jnp.float32),
            pltpu.VMEM((_NPT + 16,), jnp.float32),
            pltpu.VMEM((_NPT + 16,), jnp.float32),
            pltpu.VMEM((_SCH,), jnp.int32),
            pltpu.VMEM((_SCH + 16,), jnp.int32),
            pltpu.VMEM((_SCH + 16,), jnp.float32),
            pltpu.VMEM((_SCH + 16,), jnp.float32),
            pltpu.VMEM((_SCH + 16,), jnp.float32),
            pltpu.VMEM((_SCH + 16,), jnp.float32),
            pltpu.VMEM((_CH, 128), jnp.float32),
            pltpu.VMEM((_CH, 128), jnp.float32),
            pltpu.VMEM((128,), jnp.float32),
            pltpu.VMEM((128,), jnp.float32),
            pltpu.SemaphoreType.DMA,
            pltpu.SemaphoreType.DMA,
        ],
    )


def kernel(x, edge_index, edge_attr, batch, Wl1, bl1, Wr1, br1, We1, att1,
           bias1, Wl2, bl2, Wr2, br2, We2, att2, bias2, Wlin, blin):
    loop = jnp.arange(_N, dtype=jnp.int32)
    src2 = jnp.concatenate([edge_index[0], loop])
    dst2 = jnp.concatenate([edge_index[1], loop])
    perm = jnp.argsort(dst2).astype(jnp.int32)
    dst_s = dst2[perm]
    src_s = src2[perm]
    permc = jnp.minimum(perm, _E)

    srcs = jnp.zeros((_EPAD,), jnp.int32).at[:_E2].set(src_s)
    dsts = jnp.full((_EPAD,), _NPAD, jnp.int32).at[:_E2].set(dst_s)
    dstg = jnp.minimum(dsts, _NPAD - 1)
    permcp = jnp.zeros((_EPAD,), jnp.int32).at[:_E2].set(permc)
    rowptr = jnp.searchsorted(dsts, jnp.arange(_NPAD + 1, dtype=jnp.int32)
                              ).astype(jnp.int32)
    rowptr = jnp.concatenate(
        [rowptr, jnp.full((_NW * _NPT + _RPS - _NPAD - 1,), _E2, jnp.int32)])

    mean_row = _mean_rows(edge_attr)
    ea_ext = jnp.concatenate(
        [edge_attr, jnp.broadcast_to(mean_row, (_E1PAD - _E, 16))])
    ee1, ee2 = _edge_embed(ea_ext, We1, We2)

    xpad = jnp.zeros((_NPAD, x.shape[1]), jnp.float32).at[:_N].set(x)
    xl1, xr1 = _proj(xpad, Wl1, bl1, Wr1, br1)

    alpha1 = _alpha_kernel(2)(xl1, xr1, ee1, srcs, dstg, permcp,
                              att1.reshape(128))
    h = _agg_kernel(2, True)(xl1, alpha1, srcs, dsts, rowptr,
                             bias1.reshape(128))

    xl2, xr2 = _proj(h, Wl2, bl2, Wr2, br2)
    alpha2 = _alpha_kernel(1)(xl2, xr2, ee2, srcs, dstg, permcp,
                              att2.reshape(128))
    h2 = _agg_kernel(1, False)(xl2, alpha2, srcs, dsts, rowptr,
                               bias2.reshape(128))

    batchp = jnp.full((_NPAD,), _G, jnp.int32).at[:_N].set(batch)
    return _pool_final(h2, batchp, Wlin, blin)
